# baseline (speedup 1.0000x reference)
import functools

import jax
import jax.numpy as jnp
from jax.experimental import pallas as pl
from jax.experimental.pallas import tpu as pltpu

_F32 = jnp.float32
_BF16 = jnp.bfloat16
_FP8 = jnp.float8_e4m3fn

_VMEM_LIMIT = 50 << 20


def _proj_kernel(x_ref, wl_ref, wr_ref, c0_ref, rs_ref, xwl_ref, self_ref):
    xb = x_ref[...].astype(_BF16)
    xwl_ref[...] = jnp.dot(xb, wl_ref[...],
                           preferred_element_type=_F32).astype(_BF16)
    c0 = c0_ref[0:1, :]
    c1 = c0_ref[1:2, :]
    self_ref[...] = (jnp.dot(xb, wr_ref[...], preferred_element_type=_F32)
                     + c0 + rs_ref[...] * c1)


def _layer1_kernel(a_ref, xwl_ref, invdeg_ref, self_ref, wl2_ref, wr2_ref,
                   b2_ref, h1w_ref, h1r_ref, acc_ref):
    k = pl.program_id(1)

    @pl.when(k == 0)
    def _():
        acc_ref[...] = jnp.zeros_like(acc_ref)

    acc_ref[...] += jnp.dot(a_ref[...], xwl_ref[...],
                            preferred_element_type=_F32)

    @pl.when(k == pl.num_programs(1) - 1)
    def _():
        h1 = jnp.maximum(acc_ref[...] * invdeg_ref[...] + self_ref[...], 0.0)
        h1b = h1.astype(_BF16)
        h1w_ref[...] = jnp.dot(h1b, wl2_ref[...],
                               preferred_element_type=_F32).astype(_BF16)
        h1r_ref[...] = (jnp.dot(h1b, wr2_ref[...],
                                preferred_element_type=_F32) + b2_ref[...])


def _layer2_kernel(a_ref, h1w_ref, invdeg_ref, self_ref, wo_ref, bo_ref,
                   o_ref, acc_ref):
    k = pl.program_id(1)

    @pl.when(k == 0)
    def _():
        acc_ref[...] = jnp.zeros_like(acc_ref)

    acc_ref[...] += jnp.dot(a_ref[...], h1w_ref[...],
                            preferred_element_type=_F32)

    @pl.when(k == pl.num_programs(1) - 1)
    def _():
        h2 = jnp.maximum(acc_ref[...] * invdeg_ref[...] + self_ref[...], 0.0)
        o_ref[...] = (jnp.sum(h2 * wo_ref[...], axis=-1, keepdims=True)
                      + bo_ref[...])


@functools.partial(jax.jit, static_argnames=("num_nodes",))
def _forward(params, x, edge_index, t, num_nodes):
    in_dim = x.shape[1]
    hidden = params["wt1"].shape[0]
    n = num_nodes
    tile_m, tile_k = min(1024, n), min(4096, n)
    grid = (n // tile_m, n // tile_k)

    te = params["embed"][t[0]][None, :]
    th = te @ params["wt1"] + params["bt1"]
    th = th * jax.nn.sigmoid(th)
    temb = th @ params["wt2"] + params["bt2"]

    wl1x, wl1t = params["wl1"][:in_dim], params["wl1"][in_dim:]
    wr1x, wr1t = params["wr1"][:in_dim], params["wr1"][in_dim:]
    c0 = params["bl1"] + temb @ wr1t
    c1 = temb @ wl1t
    c01 = jnp.concatenate([c0, c1], axis=0)

    src, dst = edge_index[0], edge_index[1]
    deg = jnp.zeros((n,), _F32).at[dst].add(jnp.ones(src.shape[0], _F32))
    a_ind = (jnp.zeros((n, n), _FP8)
             .at[dst, src].add(jnp.ones(src.shape[0], _FP8)))
    invdeg = (1.0 / jnp.maximum(deg, 1.0))[:, None]
    rowsum = (deg > 0).astype(_F32)[:, None]

    row = lambda r, c: pl.BlockSpec((r, c), lambda i, k: (i, 0))
    panel = lambda r, c: pl.BlockSpec((r, c), lambda i, k: (k, 0))
    const = lambda shape: pl.BlockSpec(shape, lambda i, k: (0, 0))
    a_spec = pl.BlockSpec((tile_m, tile_k), lambda i, k: (i, k))
    cparams = pltpu.CompilerParams(
        dimension_semantics=("parallel", "arbitrary"),
        vmem_limit_bytes=_VMEM_LIMIT)

    proj_m = min(2048, n)
    xwl, self1 = pl.pallas_call(
        _proj_kernel,
        out_shape=(jax.ShapeDtypeStruct((n, hidden), _BF16),
                   jax.ShapeDtypeStruct((n, hidden), _F32)),
        grid=(n // proj_m,),
        in_specs=[
            pl.BlockSpec((proj_m, in_dim), lambda i: (i, 0)),
            pl.BlockSpec((in_dim, hidden), lambda i: (0, 0)),
            pl.BlockSpec((in_dim, hidden), lambda i: (0, 0)),
            pl.BlockSpec((2, hidden), lambda i: (0, 0)),
            pl.BlockSpec((proj_m, 1), lambda i: (i, 0)),
        ],
        out_specs=[pl.BlockSpec((proj_m, hidden), lambda i: (i, 0)),
                   pl.BlockSpec((proj_m, hidden), lambda i: (i, 0))],
        compiler_params=pltpu.CompilerParams(
            dimension_semantics=("parallel",),
            vmem_limit_bytes=_VMEM_LIMIT),
    )(x, wl1x.astype(_BF16), wr1x.astype(_BF16), c01, rowsum)

    wl2_bf = params["wl2"].astype(_BF16)
    wr2_bf = params["wr2"].astype(_BF16)

    h1w, h1r = pl.pallas_call(
        _layer1_kernel,
        out_shape=(jax.ShapeDtypeStruct((n, hidden), _BF16),
                   jax.ShapeDtypeStruct((n, hidden), _F32)),
        grid=grid,
        in_specs=[
            a_spec,
            panel(tile_k, hidden),
            row(tile_m, 1),
            row(tile_m, hidden),
            const((hidden, hidden)),
            const((hidden, hidden)),
            const((1, hidden)),
        ],
        out_specs=[row(tile_m, hidden), row(tile_m, hidden)],
        scratch_shapes=[pltpu.VMEM((tile_m, hidden), _F32)],
        compiler_params=cparams,
    )(a_ind, xwl, invdeg, self1, wl2_bf, wr2_bf, params["bl2"])

    out = pl.pallas_call(
        _layer2_kernel,
        out_shape=jax.ShapeDtypeStruct((n, 1), _F32),
        grid=grid,
        in_specs=[
            a_spec,
            panel(tile_k, hidden),
            row(tile_m, 1),
            row(tile_m, hidden),
            const((1, hidden)),
            const((1, 1)),
        ],
        out_specs=row(tile_m, 1),
        scratch_shapes=[pltpu.VMEM((tile_m, hidden), _F32)],
        compiler_params=cparams,
    )(a_ind, h1w, invdeg, h1r, params["wo"].T, params["bo"])

    return out[:, 0]


def kernel(embed, wt1, bt1, wt2, bt2, wl1, bl1, wr1, wl2, bl2, wr2, wo, bo,
           x, edge_index, t):
    params = {
        "embed": embed, "wt1": wt1, "bt1": bt1, "wt2": wt2, "bt2": bt2,
        "wl1": wl1, "bl1": bl1, "wr1": wr1, "wl2": wl2, "bl2": bl2,
        "wr2": wr2, "wo": wo, "bo": bo,
    }
    return _forward(params, x, edge_index, t, num_nodes=x.shape[0])


# speedup vs baseline: 2.8412x; 2.8412x over previous
import functools

import jax
import jax.numpy as jnp
from jax.experimental import pallas as pl
from jax.experimental.pallas import tpu as pltpu

_F32 = jnp.float32
_BF16 = jnp.bfloat16
_FP8 = jnp.float8_e4m3fn

_VMEM_LIMIT = 50 << 20


def _adj_build_kernel(n_panels, sm, pw, pb_bits,
                      bounds_ref, keys_ref, out_ref, acc_ref):
    i = pl.program_id(0)
    lane = jax.lax.broadcasted_iota(jnp.int32, (1, 128), 1)
    sub_s = jax.lax.broadcasted_iota(jnp.int32, (pw, 128), 0)
    sub_d = jax.lax.broadcasted_iota(jnp.int32, (sm, 128), 0)

    for p in range(n_panels):
        pid = i * n_panels + p
        start = bounds_ref[pid]
        end = bounds_ref[pid + 1]
        base_row = start >> 7
        nch = (end - (base_row << 7) + 255) >> 8

        acc_ref[...] = jnp.zeros_like(acc_ref)

        def body(c, carry):
            row = base_row + 2 * c
            k0 = keys_ref[pl.ds(row, 1), :]
            k1 = keys_ref[pl.ds(row + 1, 1), :]
            e00 = row << 7
            v0 = ((e00 + lane) >= start) & ((e00 + lane) < end)
            v1 = ((e00 + 128 + lane) >= start) & ((e00 + 128 + lane) < end)
            sl0 = jnp.where(v0, k0 & (pw - 1), pw)
            sl1 = jnp.where(v1, k1 & (pw - 1), pw)
            dl0 = (k0 >> pb_bits) & (sm - 1)
            dl1 = (k1 >> pb_bits) & (sm - 1)
            s_oh = jnp.concatenate([(sub_s == sl0), (sub_s == sl1)],
                                   axis=1).astype(_FP8)
            d_oh = jnp.concatenate([(sub_d == dl0), (sub_d == dl1)],
                                   axis=1).astype(_FP8)
            acc_ref[...] += jax.lax.dot_general(
                d_oh, s_oh, (((1,), (1,)), ((), ())),
                preferred_element_type=_F32)
            return carry

        jax.lax.fori_loop(0, nch, body, 0)
        out_ref[:, p * pw:(p + 1) * pw] = acc_ref[...].astype(out_ref.dtype)


def _build_adjacency(src, dst, n):
    e = src.shape[0]
    sm, pw = min(256, n), min(1024, n)
    n_strips, n_panels = n // sm, n // pw
    pb_bits = (pw - 1).bit_length()
    db_bits = (sm - 1).bit_length()

    strip = dst // sm
    panel = src // pw
    key = ((((strip * n_panels) + panel) << (db_bits + pb_bits))
           | ((dst & (sm - 1)) << pb_bits) | (src & (pw - 1)))
    sk = jax.lax.sort(key)

    npairs = n_strips * n_panels
    starts = (jnp.arange(npairs + 1, dtype=jnp.int32)
              << (db_bits + pb_bits))
    bounds = jnp.searchsorted(sk, starts, side='left').astype(jnp.int32)

    rows = e // 128
    keys2d = jnp.concatenate(
        [sk, jnp.full((8 * 128,), jnp.iinfo(jnp.int32).max, jnp.int32)]
    ).reshape(rows + 8, 128)

    return pl.pallas_call(
        functools.partial(_adj_build_kernel, n_panels, sm, pw, pb_bits),
        out_shape=jax.ShapeDtypeStruct((n, n), _FP8),
        grid=(n_strips,),
        in_specs=[
            pl.BlockSpec(memory_space=pltpu.SMEM),
            pl.BlockSpec((rows + 8, 128), lambda i: (0, 0)),
        ],
        out_specs=pl.BlockSpec((sm, n), lambda i: (i, 0)),
        scratch_shapes=[pltpu.VMEM((sm, pw), _F32)],
        compiler_params=pltpu.CompilerParams(
            dimension_semantics=("parallel",),
            vmem_limit_bytes=_VMEM_LIMIT),
    )(bounds, keys2d)


def _proj_kernel(x_ref, wl_ref, wr_ref, c0_ref, rs_ref, xwl_ref, self_ref):
    xb = x_ref[...].astype(_BF16)
    xwl_ref[...] = jnp.dot(xb, wl_ref[...],
                           preferred_element_type=_F32).astype(_BF16)
    c0 = c0_ref[0:1, :]
    c1 = c0_ref[1:2, :]
    self_ref[...] = (jnp.dot(xb, wr_ref[...], preferred_element_type=_F32)
                     + c0 + rs_ref[...] * c1)


def _layer1_kernel(a_ref, xwl_ref, invdeg_ref, self_ref, wl2_ref, wr2_ref,
                   b2_ref, h1w_ref, h1r_ref, acc_ref):
    k = pl.program_id(1)

    @pl.when(k == 0)
    def _():
        acc_ref[...] = jnp.zeros_like(acc_ref)

    acc_ref[...] += jnp.dot(a_ref[...], xwl_ref[...],
                            preferred_element_type=_F32)

    @pl.when(k == pl.num_programs(1) - 1)
    def _():
        h1 = jnp.maximum(acc_ref[...] * invdeg_ref[...] + self_ref[...], 0.0)
        h1b = h1.astype(_BF16)
        h1w_ref[...] = jnp.dot(h1b, wl2_ref[...],
                               preferred_element_type=_F32).astype(_BF16)
        h1r_ref[...] = (jnp.dot(h1b, wr2_ref[...],
                                preferred_element_type=_F32) + b2_ref[...])


def _layer2_kernel(a_ref, h1w_ref, invdeg_ref, self_ref, wo_ref, bo_ref,
                   o_ref, acc_ref):
    k = pl.program_id(1)

    @pl.when(k == 0)
    def _():
        acc_ref[...] = jnp.zeros_like(acc_ref)

    acc_ref[...] += jnp.dot(a_ref[...], h1w_ref[...],
                            preferred_element_type=_F32)

    @pl.when(k == pl.num_programs(1) - 1)
    def _():
        h2 = jnp.maximum(acc_ref[...] * invdeg_ref[...] + self_ref[...], 0.0)
        o_ref[...] = (jnp.sum(h2 * wo_ref[...], axis=-1, keepdims=True)
                      + bo_ref[...])


@functools.partial(jax.jit, static_argnames=("num_nodes",))
def _forward(params, x, edge_index, t, num_nodes):
    in_dim = x.shape[1]
    hidden = params["wt1"].shape[0]
    n = num_nodes
    tile_m, tile_k = min(1024, n), min(4096, n)
    grid = (n // tile_m, n // tile_k)

    te = params["embed"][t[0]][None, :]
    th = te @ params["wt1"] + params["bt1"]
    th = th * jax.nn.sigmoid(th)
    temb = th @ params["wt2"] + params["bt2"]

    wl1x, wl1t = params["wl1"][:in_dim], params["wl1"][in_dim:]
    wr1x, wr1t = params["wr1"][:in_dim], params["wr1"][in_dim:]
    c0 = params["bl1"] + temb @ wr1t
    c1 = temb @ wl1t
    c01 = jnp.concatenate([c0, c1], axis=0)

    src, dst = edge_index[0], edge_index[1]
    deg = jnp.zeros((n,), _F32).at[dst].add(jnp.ones(src.shape[0], _F32))
    invdeg = (1.0 / jnp.maximum(deg, 1.0))[:, None]
    rowsum = (deg > 0).astype(_F32)[:, None]

    a_ind = _build_adjacency(src, dst, n)

    row = lambda r, c: pl.BlockSpec((r, c), lambda i, k: (i, 0))
    panel = lambda r, c: pl.BlockSpec((r, c), lambda i, k: (k, 0))
    const = lambda shape: pl.BlockSpec(shape, lambda i, k: (0, 0))
    a_spec = pl.BlockSpec((tile_m, tile_k), lambda i, k: (i, k))
    cparams = pltpu.CompilerParams(
        dimension_semantics=("parallel", "arbitrary"),
        vmem_limit_bytes=_VMEM_LIMIT)

    proj_m = min(2048, n)
    xwl, self1 = pl.pallas_call(
        _proj_kernel,
        out_shape=(jax.ShapeDtypeStruct((n, hidden), _BF16),
                   jax.ShapeDtypeStruct((n, hidden), _F32)),
        grid=(n // proj_m,),
        in_specs=[
            pl.BlockSpec((proj_m, in_dim), lambda i: (i, 0)),
            pl.BlockSpec((in_dim, hidden), lambda i: (0, 0)),
            pl.BlockSpec((in_dim, hidden), lambda i: (0, 0)),
            pl.BlockSpec((2, hidden), lambda i: (0, 0)),
            pl.BlockSpec((proj_m, 1), lambda i: (i, 0)),
        ],
        out_specs=[pl.BlockSpec((proj_m, hidden), lambda i: (i, 0)),
                   pl.BlockSpec((proj_m, hidden), lambda i: (i, 0))],
        compiler_params=pltpu.CompilerParams(
            dimension_semantics=("parallel",),
            vmem_limit_bytes=_VMEM_LIMIT),
    )(x, wl1x.astype(_BF16), wr1x.astype(_BF16), c01, rowsum)

    wl2_bf = params["wl2"].astype(_BF16)
    wr2_bf = params["wr2"].astype(_BF16)

    h1w, h1r = pl.pallas_call(
        _layer1_kernel,
        out_shape=(jax.ShapeDtypeStruct((n, hidden), _BF16),
                   jax.ShapeDtypeStruct((n, hidden), _F32)),
        grid=grid,
        in_specs=[
            a_spec,
            panel(tile_k, hidden),
            row(tile_m, 1),
            row(tile_m, hidden),
            const((hidden, hidden)),
            const((hidden, hidden)),
            const((1, hidden)),
        ],
        out_specs=[row(tile_m, hidden), row(tile_m, hidden)],
        scratch_shapes=[pltpu.VMEM((tile_m, hidden), _F32)],
        compiler_params=cparams,
    )(a_ind, xwl, invdeg, self1, wl2_bf, wr2_bf, params["bl2"])

    out = pl.pallas_call(
        _layer2_kernel,
        out_shape=jax.ShapeDtypeStruct((n, 1), _F32),
        grid=grid,
        in_specs=[
            a_spec,
            panel(tile_k, hidden),
            row(tile_m, 1),
            row(tile_m, hidden),
            const((1, hidden)),
            const((1, 1)),
        ],
        out_specs=row(tile_m, 1),
        scratch_shapes=[pltpu.VMEM((tile_m, hidden), _F32)],
        compiler_params=cparams,
    )(a_ind, h1w, invdeg, h1r, params["wo"].T, params["bo"])

    return out[:, 0]


def kernel(embed, wt1, bt1, wt2, bt2, wl1, bl1, wr1, wl2, bl2, wr2, wo, bo,
           x, edge_index, t):
    params = {
        "embed": embed, "wt1": wt1, "bt1": bt1, "wt2": wt2, "bt2": bt2,
        "wl1": wl1, "bl1": bl1, "wr1": wr1, "wl2": wl2, "bl2": bl2,
        "wr2": wr2, "wo": wo, "bo": bo,
    }
    return _forward(params, x, edge_index, t, num_nodes=x.shape[0])


# speedup vs baseline: 2.9602x; 1.0419x over previous
import functools

import jax
import jax.numpy as jnp
from jax.experimental import pallas as pl
from jax.experimental.pallas import tpu as pltpu

_F32 = jnp.float32
_BF16 = jnp.bfloat16
_FP8 = jnp.float8_e4m3fn

_VMEM_LIMIT = 50 << 20


def _adj_build_kernel(n_panels, sm, pw, pb_bits,
                      bounds_ref, keys_ref, out_ref, deg_ref, acc_ref):
    i = pl.program_id(0)
    lane = jax.lax.broadcasted_iota(jnp.int32, (1, 128), 1)
    sub_s = jax.lax.broadcasted_iota(jnp.int32, (pw, 128), 0)
    sub_d = jax.lax.broadcasted_iota(jnp.int32, (sm, 128), 0)
    dsum = jnp.zeros((sm, 1), _F32)

    for p in range(n_panels):
        pid = i * n_panels + p
        start = bounds_ref[pid]
        end = bounds_ref[pid + 1]
        base_row = start >> 7
        nch = (end - (base_row << 7) + 255) >> 8

        acc_ref[...] = jnp.zeros_like(acc_ref)

        def body(c, carry):
            row = base_row + 2 * c
            k0 = keys_ref[pl.ds(row, 1), :]
            k1 = keys_ref[pl.ds(row + 1, 1), :]
            e00 = row << 7
            v0 = ((e00 + lane) >= start) & ((e00 + lane) < end)
            v1 = ((e00 + 128 + lane) >= start) & ((e00 + 128 + lane) < end)
            sl0 = jnp.where(v0, k0 & (pw - 1), pw)
            sl1 = jnp.where(v1, k1 & (pw - 1), pw)
            dl0 = (k0 >> pb_bits) & (sm - 1)
            dl1 = (k1 >> pb_bits) & (sm - 1)
            s_oh = jnp.concatenate([(sub_s == sl0), (sub_s == sl1)],
                                   axis=1).astype(_FP8)
            d_oh = jnp.concatenate([(sub_d == dl0), (sub_d == dl1)],
                                   axis=1).astype(_FP8)
            acc_ref[...] += jax.lax.dot_general(
                d_oh, s_oh, (((1,), (1,)), ((), ())),
                preferred_element_type=_F32)
            return carry

        jax.lax.fori_loop(0, nch, body, 0)
        out_ref[:, p * pw:(p + 1) * pw] = acc_ref[...].astype(out_ref.dtype)
        dsum = dsum + jnp.sum(acc_ref[...], axis=1, keepdims=True)

    deg_ref[...] = dsum


def _build_adjacency(src, dst, n):
    e = src.shape[0]
    sm, pw = min(256, n), min(1024, n)
    n_strips, n_panels = n // sm, n // pw
    pb_bits = (pw - 1).bit_length()
    db_bits = (sm - 1).bit_length()

    strip = dst // sm
    panel = src // pw
    key = ((((strip * n_panels) + panel) << (db_bits + pb_bits))
           | ((dst & (sm - 1)) << pb_bits) | (src & (pw - 1)))
    sk = jax.lax.sort(key)

    npairs = n_strips * n_panels
    starts = (jnp.arange(npairs + 1, dtype=jnp.int32)
              << (db_bits + pb_bits))
    bounds = jnp.searchsorted(sk, starts, side='left').astype(jnp.int32)

    rows = e // 128
    keys2d = jnp.concatenate(
        [sk, jnp.full((8 * 128,), jnp.iinfo(jnp.int32).max, jnp.int32)]
    ).reshape(rows + 8, 128)

    return pl.pallas_call(
        functools.partial(_adj_build_kernel, n_panels, sm, pw, pb_bits),
        out_shape=(jax.ShapeDtypeStruct((n, n), _FP8),
                   jax.ShapeDtypeStruct((n, 1), _F32)),
        grid=(n_strips,),
        in_specs=[
            pl.BlockSpec(memory_space=pltpu.SMEM),
            pl.BlockSpec((rows + 8, 128), lambda i: (0, 0)),
        ],
        out_specs=[pl.BlockSpec((sm, n), lambda i: (i, 0)),
                   pl.BlockSpec((sm, 1), lambda i: (i, 0))],
        scratch_shapes=[pltpu.VMEM((sm, pw), _F32)],
        compiler_params=pltpu.CompilerParams(
            dimension_semantics=("parallel",),
            vmem_limit_bytes=_VMEM_LIMIT),
    )(bounds, keys2d)


def _proj_kernel(x_ref, wl_ref, wr_ref, c0_ref, rs_ref, xwl_ref, self_ref):
    xb = x_ref[...].astype(_BF16)
    xwl_ref[...] = jnp.dot(xb, wl_ref[...],
                           preferred_element_type=_F32).astype(_BF16)
    c0 = c0_ref[0:1, :]
    c1 = c0_ref[1:2, :]
    self_ref[...] = (jnp.dot(xb, wr_ref[...], preferred_element_type=_F32)
                     + c0 + rs_ref[...] * c1)


def _layer1_kernel(a_ref, xwl_ref, invdeg_ref, self_ref, wl2_ref, wr2_ref,
                   b2_ref, h1w_ref, h1r_ref, acc_ref):
    k = pl.program_id(1)

    @pl.when(k == 0)
    def _():
        acc_ref[...] = jnp.zeros_like(acc_ref)

    acc_ref[...] += jax.lax.dot_general(
        a_ref[...], xwl_ref[...], (((1,), (0,)), ((), ())),
        preferred_element_type=_F32)

    @pl.when(k == pl.num_programs(1) - 1)
    def _():
        h1 = jnp.maximum(acc_ref[...] * invdeg_ref[...] + self_ref[...], 0.0)
        h1b = h1.astype(_BF16)
        h1w_ref[...] = jnp.dot(h1b, wl2_ref[...],
                               preferred_element_type=_F32).astype(_BF16)
        h1r_ref[...] = (jnp.dot(h1b, wr2_ref[...],
                                preferred_element_type=_F32) + b2_ref[...])


def _layer2_kernel(a_ref, h1w_ref, invdeg_ref, self_ref, wo_ref, bo_ref,
                   o_ref, acc_ref):
    k = pl.program_id(1)

    @pl.when(k == 0)
    def _():
        acc_ref[...] = jnp.zeros_like(acc_ref)

    acc_ref[...] += jax.lax.dot_general(
        a_ref[...], h1w_ref[...], (((1,), (0,)), ((), ())),
        preferred_element_type=_F32)

    @pl.when(k == pl.num_programs(1) - 1)
    def _():
        h2 = jnp.maximum(acc_ref[...] * invdeg_ref[...] + self_ref[...], 0.0)
        o_ref[...] = (jnp.sum(h2 * wo_ref[...], axis=-1, keepdims=True)
                      + bo_ref[...])


@functools.partial(jax.jit, static_argnames=("num_nodes",))
def _forward(params, x, edge_index, t, num_nodes):
    in_dim = x.shape[1]
    hidden = params["wt1"].shape[0]
    n = num_nodes
    tile_m, tile_k = min(1024, n), min(4096, n)
    grid = (n // tile_m, n // tile_k)

    te = params["embed"][t[0]][None, :]
    th = te @ params["wt1"] + params["bt1"]
    th = th * jax.nn.sigmoid(th)
    temb = th @ params["wt2"] + params["bt2"]

    wl1x, wl1t = params["wl1"][:in_dim], params["wl1"][in_dim:]
    wr1x, wr1t = params["wr1"][:in_dim], params["wr1"][in_dim:]
    c0 = params["bl1"] + temb @ wr1t
    c1 = temb @ wl1t
    c01 = jnp.concatenate([c0, c1], axis=0)

    src, dst = edge_index[0], edge_index[1]
    a_ind, deg = _build_adjacency(src, dst, n)
    invdeg = 1.0 / jnp.maximum(deg, 1.0)
    rowsum = (deg > 0).astype(_F32)

    row = lambda r, c: pl.BlockSpec((r, c), lambda i, k: (i, 0))
    panel = lambda r, c: pl.BlockSpec((r, c), lambda i, k: (k, 0))
    const = lambda shape: pl.BlockSpec(shape, lambda i, k: (0, 0))
    a_spec = pl.BlockSpec((tile_m, tile_k), lambda i, k: (i, k))
    cparams = pltpu.CompilerParams(
        dimension_semantics=("parallel", "arbitrary"),
        vmem_limit_bytes=_VMEM_LIMIT)

    proj_m = min(2048, n)
    xwl, self1 = pl.pallas_call(
        _proj_kernel,
        out_shape=(jax.ShapeDtypeStruct((n, hidden), _BF16),
                   jax.ShapeDtypeStruct((n, hidden), _F32)),
        grid=(n // proj_m,),
        in_specs=[
            pl.BlockSpec((proj_m, in_dim), lambda i: (i, 0)),
            pl.BlockSpec((in_dim, hidden), lambda i: (0, 0)),
            pl.BlockSpec((in_dim, hidden), lambda i: (0, 0)),
            pl.BlockSpec((2, hidden), lambda i: (0, 0)),
            pl.BlockSpec((proj_m, 1), lambda i: (i, 0)),
        ],
        out_specs=[pl.BlockSpec((proj_m, hidden), lambda i: (i, 0)),
                   pl.BlockSpec((proj_m, hidden), lambda i: (i, 0))],
        compiler_params=pltpu.CompilerParams(
            dimension_semantics=("parallel",),
            vmem_limit_bytes=_VMEM_LIMIT),
    )(x, wl1x.astype(_BF16), wr1x.astype(_BF16), c01, rowsum)

    wl2_bf = params["wl2"].astype(_BF16)
    wr2_bf = params["wr2"].astype(_BF16)

    h1w, h1r = pl.pallas_call(
        _layer1_kernel,
        out_shape=(jax.ShapeDtypeStruct((n, hidden), _BF16),
                   jax.ShapeDtypeStruct((n, hidden), _F32)),
        grid=grid,
        in_specs=[
            a_spec,
            panel(tile_k, hidden),
            row(tile_m, 1),
            row(tile_m, hidden),
            const((hidden, hidden)),
            const((hidden, hidden)),
            const((1, hidden)),
        ],
        out_specs=[row(tile_m, hidden), row(tile_m, hidden)],
        scratch_shapes=[pltpu.VMEM((tile_m, hidden), _F32)],
        compiler_params=cparams,
    )(a_ind, xwl, invdeg, self1, wl2_bf, wr2_bf, params["bl2"])

    out = pl.pallas_call(
        _layer2_kernel,
        out_shape=jax.ShapeDtypeStruct((n, 1), _F32),
        grid=grid,
        in_specs=[
            a_spec,
            panel(tile_k, hidden),
            row(tile_m, 1),
            row(tile_m, hidden),
            const((1, hidden)),
            const((1, 1)),
        ],
        out_specs=row(tile_m, 1),
        scratch_shapes=[pltpu.VMEM((tile_m, hidden), _F32)],
        compiler_params=cparams,
    )(a_ind, h1w, invdeg, h1r, params["wo"].T, params["bo"])

    return out[:, 0]


def kernel(embed, wt1, bt1, wt2, bt2, wl1, bl1, wr1, wl2, bl2, wr2, wo, bo,
           x, edge_index, t):
    params = {
        "embed": embed, "wt1": wt1, "bt1": bt1, "wt2": wt2, "bt2": bt2,
        "wl1": wl1, "bl1": bl1, "wr1": wr1, "wl2": wl2, "bl2": bl2,
        "wr2": wr2, "wo": wo, "bo": bo,
    }
    return _forward(params, x, edge_index, t, num_nodes=x.shape[0])


# speedup vs baseline: 3.3036x; 1.1160x over previous
import functools

import jax
import jax.numpy as jnp
from jax.experimental import pallas as pl
from jax.experimental.pallas import tpu as pltpu

_F32 = jnp.float32
_BF16 = jnp.bfloat16
_FP8 = jnp.float8_e4m3fn

_VMEM_LIMIT = 50 << 20


def _adj_build_kernel(n_panels, sm, pw, pb_bits,
                      bounds_ref, keys_ref, out_ref, acc_ref):
    i = pl.program_id(0)
    lane = jax.lax.broadcasted_iota(jnp.int32, (1, 128), 1)
    sub_s = jax.lax.broadcasted_iota(jnp.int32, (pw, 128), 0)
    sub_d = jax.lax.broadcasted_iota(jnp.int32, (sm, 128), 0)

    def onehots(row, start, end):
        k = keys_ref[pl.ds(row, 1), :]
        e0 = row << 7
        valid = ((e0 + lane) >= start) & ((e0 + lane) < end)
        sl = jnp.where(valid, k & (pw - 1), pw)
        dl = (k >> pb_bits) & (sm - 1)
        return (sub_s == sl), (sub_d == dl)

    for p in range(n_panels):
        pid = i * n_panels + p
        start = bounds_ref[pid]
        end = bounds_ref[pid + 1]
        base_row = start >> 7

        parts = [onehots(base_row + j, start, end) for j in range(4)]
        s_oh = jnp.concatenate([s for s, _ in parts], axis=1).astype(_FP8)
        d_oh = jnp.concatenate([d for _, d in parts], axis=1).astype(_FP8)
        res = jax.lax.dot_general(d_oh, s_oh, (((1,), (1,)), ((), ())),
                                  preferred_element_type=_F32)
        out_ref[:, p * pw:(p + 1) * pw] = res.astype(out_ref.dtype)

        @pl.when(end > (base_row << 7) + 512)
        def _():
            acc_ref[...] = jnp.zeros_like(acc_ref)
            nch = (end - (base_row << 7) + 255) >> 8

            def body(c, carry):
                row = base_row + 2 * c
                s0, d0 = onehots(row, start, end)
                s1, d1 = onehots(row + 1, start, end)
                s2 = jnp.concatenate([s0, s1], axis=1).astype(_FP8)
                d2 = jnp.concatenate([d0, d1], axis=1).astype(_FP8)
                acc_ref[...] += jax.lax.dot_general(
                    d2, s2, (((1,), (1,)), ((), ())),
                    preferred_element_type=_F32)
                return carry

            jax.lax.fori_loop(2, nch, body, 0)
            total = out_ref[:, p * pw:(p + 1) * pw].astype(_F32) + acc_ref[...]
            out_ref[:, p * pw:(p + 1) * pw] = total.astype(out_ref.dtype)


def _build_adjacency(src, dst, n):
    e = src.shape[0]
    sm, pw = min(256, n), min(1024, n)
    n_strips, n_panels = n // sm, n // pw
    pb_bits = (pw - 1).bit_length()
    db_bits = (sm - 1).bit_length()

    strip = dst // sm
    panel = src // pw
    key = ((((strip * n_panels) + panel) << (db_bits + pb_bits))
           | ((dst & (sm - 1)) << pb_bits) | (src & (pw - 1)))
    sk = jax.lax.sort(key)

    npairs = n_strips * n_panels
    starts = (jnp.arange(npairs + 1, dtype=jnp.int32)
              << (db_bits + pb_bits))
    bounds = jnp.searchsorted(sk, starts, side='left').astype(jnp.int32)

    rows = e // 128
    keys2d = jnp.concatenate(
        [sk, jnp.full((8 * 128,), jnp.iinfo(jnp.int32).max, jnp.int32)]
    ).reshape(rows + 8, 128)

    return pl.pallas_call(
        functools.partial(_adj_build_kernel, n_panels, sm, pw, pb_bits),
        out_shape=jax.ShapeDtypeStruct((n, n), _FP8),
        grid=(n_strips,),
        in_specs=[
            pl.BlockSpec(memory_space=pltpu.SMEM),
            pl.BlockSpec((rows + 8, 128), lambda i: (0, 0)),
        ],
        out_specs=pl.BlockSpec((sm, n), lambda i: (i, 0)),
        scratch_shapes=[pltpu.VMEM((sm, pw), _F32)],
        compiler_params=pltpu.CompilerParams(
            dimension_semantics=("parallel",),
            vmem_limit_bytes=_VMEM_LIMIT),
    )(bounds, keys2d)


def _proj_kernel(x_ref, wl_ref, wr_ref, c0_ref, rs_ref, xwl_ref, self_ref):
    xb = x_ref[...].astype(_BF16)
    xwl_ref[...] = jnp.dot(xb, wl_ref[...],
                           preferred_element_type=_F32).astype(_BF16)
    c0 = c0_ref[0:1, :]
    c1 = c0_ref[1:2, :]
    self_ref[...] = (jnp.dot(xb, wr_ref[...], preferred_element_type=_F32)
                     + c0 + rs_ref[...] * c1)


def _layer1_kernel(a_ref, xwl_ref, invdeg_ref, self_ref, wl2_ref, wr2_ref,
                   b2_ref, h1w_ref, h1r_ref, acc_ref):
    k = pl.program_id(1)

    @pl.when(k == 0)
    def _():
        acc_ref[...] = jnp.zeros_like(acc_ref)

    acc_ref[...] += jax.lax.dot_general(
        a_ref[...], xwl_ref[...], (((1,), (0,)), ((), ())),
        preferred_element_type=_F32)

    @pl.when(k == pl.num_programs(1) - 1)
    def _():
        h1 = jnp.maximum(acc_ref[...] * invdeg_ref[...] + self_ref[...], 0.0)
        h1b = h1.astype(_BF16)
        h1w_ref[...] = jnp.dot(h1b, wl2_ref[...],
                               preferred_element_type=_F32).astype(_BF16)
        h1r_ref[...] = (jnp.dot(h1b, wr2_ref[...],
                                preferred_element_type=_F32) + b2_ref[...])


def _layer2_kernel(a_ref, h1w_ref, invdeg_ref, self_ref, wo_ref, bo_ref,
                   o_ref, acc_ref):
    k = pl.program_id(1)

    @pl.when(k == 0)
    def _():
        acc_ref[...] = jnp.zeros_like(acc_ref)

    acc_ref[...] += jax.lax.dot_general(
        a_ref[...], h1w_ref[...], (((1,), (0,)), ((), ())),
        preferred_element_type=_F32)

    @pl.when(k == pl.num_programs(1) - 1)
    def _():
        h2 = jnp.maximum(acc_ref[...] * invdeg_ref[...] + self_ref[...], 0.0)
        o_ref[...] = (jnp.sum(h2 * wo_ref[...], axis=-1, keepdims=True)
                      + bo_ref[...])


@functools.partial(jax.jit, static_argnames=("num_nodes",))
def _forward(params, x, edge_index, t, num_nodes):
    in_dim = x.shape[1]
    hidden = params["wt1"].shape[0]
    n = num_nodes
    tile_m, tile_k = min(1024, n), min(4096, n)
    grid = (n // tile_m, n // tile_k)

    te = params["embed"][t[0]][None, :]
    th = te @ params["wt1"] + params["bt1"]
    th = th * jax.nn.sigmoid(th)
    temb = th @ params["wt2"] + params["bt2"]

    wl1x, wl1t = params["wl1"][:in_dim], params["wl1"][in_dim:]
    wr1x, wr1t = params["wr1"][:in_dim], params["wr1"][in_dim:]
    c0 = params["bl1"] + temb @ wr1t
    c1 = temb @ wl1t
    c01 = jnp.concatenate([c0, c1], axis=0)

    src, dst = edge_index[0], edge_index[1]
    a_ind = _build_adjacency(src, dst, n)
    deg = jnp.zeros((n,), _F32).at[dst].add(jnp.ones(src.shape[0], _F32))
    invdeg = (1.0 / jnp.maximum(deg, 1.0))[:, None]
    rowsum = (deg > 0).astype(_F32)[:, None]

    row = lambda r, c: pl.BlockSpec((r, c), lambda i, k: (i, 0))
    panel = lambda r, c: pl.BlockSpec((r, c), lambda i, k: (k, 0))
    const = lambda shape: pl.BlockSpec(shape, lambda i, k: (0, 0))
    a_spec = pl.BlockSpec((tile_m, tile_k), lambda i, k: (i, k))
    cparams = pltpu.CompilerParams(
        dimension_semantics=("parallel", "arbitrary"),
        vmem_limit_bytes=_VMEM_LIMIT)

    proj_m = min(2048, n)
    xwl, self1 = pl.pallas_call(
        _proj_kernel,
        out_shape=(jax.ShapeDtypeStruct((n, hidden), _BF16),
                   jax.ShapeDtypeStruct((n, hidden), _F32)),
        grid=(n // proj_m,),
        in_specs=[
            pl.BlockSpec((proj_m, in_dim), lambda i: (i, 0)),
            pl.BlockSpec((in_dim, hidden), lambda i: (0, 0)),
            pl.BlockSpec((in_dim, hidden), lambda i: (0, 0)),
            pl.BlockSpec((2, hidden), lambda i: (0, 0)),
            pl.BlockSpec((proj_m, 1), lambda i: (i, 0)),
        ],
        out_specs=[pl.BlockSpec((proj_m, hidden), lambda i: (i, 0)),
                   pl.BlockSpec((proj_m, hidden), lambda i: (i, 0))],
        compiler_params=pltpu.CompilerParams(
            dimension_semantics=("parallel",),
            vmem_limit_bytes=_VMEM_LIMIT),
    )(x, wl1x.astype(_BF16), wr1x.astype(_BF16), c01, rowsum)

    wl2_bf = params["wl2"].astype(_BF16)
    wr2_bf = params["wr2"].astype(_BF16)

    h1w, h1r = pl.pallas_call(
        _layer1_kernel,
        out_shape=(jax.ShapeDtypeStruct((n, hidden), _BF16),
                   jax.ShapeDtypeStruct((n, hidden), _F32)),
        grid=grid,
        in_specs=[
            a_spec,
            panel(tile_k, hidden),
            row(tile_m, 1),
            row(tile_m, hidden),
            const((hidden, hidden)),
            const((hidden, hidden)),
            const((1, hidden)),
        ],
        out_specs=[row(tile_m, hidden), row(tile_m, hidden)],
        scratch_shapes=[pltpu.VMEM((tile_m, hidden), _F32)],
        compiler_params=cparams,
    )(a_ind, xwl, invdeg, self1, wl2_bf, wr2_bf, params["bl2"])

    out = pl.pallas_call(
        _layer2_kernel,
        out_shape=jax.ShapeDtypeStruct((n, 1), _F32),
        grid=grid,
        in_specs=[
            a_spec,
            panel(tile_k, hidden),
            row(tile_m, 1),
            row(tile_m, hidden),
            const((1, hidden)),
            const((1, 1)),
        ],
        out_specs=row(tile_m, 1),
        scratch_shapes=[pltpu.VMEM((tile_m, hidden), _F32)],
        compiler_params=cparams,
    )(a_ind, h1w, invdeg, h1r, params["wo"].T, params["bo"])

    return out[:, 0]


def kernel(embed, wt1, bt1, wt2, bt2, wl1, bl1, wr1, wl2, bl2, wr2, wo, bo,
           x, edge_index, t):
    params = {
        "embed": embed, "wt1": wt1, "bt1": bt1, "wt2": wt2, "bt2": bt2,
        "wl1": wl1, "bl1": bl1, "wr1": wr1, "wl2": wl2, "bl2": bl2,
        "wr2": wr2, "wo": wo, "bo": bo,
    }
    return _forward(params, x, edge_index, t, num_nodes=x.shape[0])


# speedup vs baseline: 3.3273x; 1.0072x over previous
import functools

import jax
import jax.numpy as jnp
from jax.experimental import pallas as pl
from jax.experimental.pallas import tpu as pltpu

_F32 = jnp.float32
_BF16 = jnp.bfloat16
_FP8 = jnp.float8_e4m3fn

_VMEM_LIMIT = 50 << 20


def _adj_build_kernel(n_panels, sm, pw, pb_bits,
                      bounds_ref, keys_ref, out_ref, acc_ref):
    i = pl.program_id(0)
    lane = jax.lax.broadcasted_iota(jnp.int32, (1, 128), 1)
    sub_s = jax.lax.broadcasted_iota(jnp.int32, (pw, 128), 0)
    sub_d = jax.lax.broadcasted_iota(jnp.int32, (sm, 128), 0)

    def onehots(row, start, end):
        k = keys_ref[pl.ds(row, 1), :]
        e0 = row << 7
        valid = ((e0 + lane) >= start) & ((e0 + lane) < end)
        sl = jnp.where(valid, k & (pw - 1), pw)
        dl = (k >> pb_bits) & (sm - 1)
        return (sub_s == sl), (sub_d == dl)

    for p in range(n_panels):
        pid = i * n_panels + p
        start = bounds_ref[pid]
        end = bounds_ref[pid + 1]
        base_row = start >> 7

        parts = [onehots(base_row + j, start, end) for j in range(4)]
        s_oh = jnp.concatenate([s for s, _ in parts], axis=1).astype(_FP8)
        d_oh = jnp.concatenate([d for _, d in parts], axis=1).astype(_FP8)
        res = jax.lax.dot_general(d_oh, s_oh, (((1,), (1,)), ((), ())),
                                  preferred_element_type=_F32)
        out_ref[:, p * pw:(p + 1) * pw] = res.astype(out_ref.dtype)

        @pl.when(end > (base_row << 7) + 512)
        def _():
            acc_ref[...] = jnp.zeros_like(acc_ref)
            nch = (end - (base_row << 7) + 255) >> 8

            def body(c, carry):
                row = base_row + 2 * c
                s0, d0 = onehots(row, start, end)
                s1, d1 = onehots(row + 1, start, end)
                s2 = jnp.concatenate([s0, s1], axis=1).astype(_FP8)
                d2 = jnp.concatenate([d0, d1], axis=1).astype(_FP8)
                acc_ref[...] += jax.lax.dot_general(
                    d2, s2, (((1,), (1,)), ((), ())),
                    preferred_element_type=_F32)
                return carry

            jax.lax.fori_loop(2, nch, body, 0)
            total = out_ref[:, p * pw:(p + 1) * pw].astype(_F32) + acc_ref[...]
            out_ref[:, p * pw:(p + 1) * pw] = total.astype(out_ref.dtype)


def _searchsorted_rows(sk, starts):
    e = sk.shape[0]
    rows = e // 128
    sk2d = sk.reshape(rows, 128)
    heads = sk2d[:, 0]
    c = jnp.sum(heads[None, :] < starts[:, None], axis=1)
    row = jnp.maximum(c - 1, 0)
    oh = (row[:, None] == jnp.arange(rows)[None, :]).astype(_F32)
    hi = (sk2d >> 14).astype(_F32)
    lo = (sk2d & 0x3FFF).astype(_F32)
    vals = (jnp.dot(oh, hi).astype(jnp.int32) << 14) | \
        jnp.dot(oh, lo).astype(jnp.int32)
    cnt = jnp.sum(vals < starts[:, None], axis=1)
    return (row * 128 + cnt).astype(jnp.int32)


def _build_adjacency(src, dst, n):
    e = src.shape[0]
    sm, pw = min(256, n), min(1024, n)
    n_strips, n_panels = n // sm, n // pw
    pb_bits = (pw - 1).bit_length()
    db_bits = (sm - 1).bit_length()

    strip = dst // sm
    panel = src // pw
    key = ((((strip * n_panels) + panel) << (db_bits + pb_bits))
           | ((dst & (sm - 1)) << pb_bits) | (src & (pw - 1)))
    sk = jax.lax.sort(key)

    npairs = n_strips * n_panels
    starts = (jnp.arange(npairs + 1, dtype=jnp.int32)
              << (db_bits + pb_bits))
    bounds = _searchsorted_rows(sk, starts)

    rows = e // 128
    keys2d = jnp.concatenate(
        [sk, jnp.full((8 * 128,), jnp.iinfo(jnp.int32).max, jnp.int32)]
    ).reshape(rows + 8, 128)

    return pl.pallas_call(
        functools.partial(_adj_build_kernel, n_panels, sm, pw, pb_bits),
        out_shape=jax.ShapeDtypeStruct((n, n), _FP8),
        grid=(n_strips,),
        in_specs=[
            pl.BlockSpec(memory_space=pltpu.SMEM),
            pl.BlockSpec((rows + 8, 128), lambda i: (0, 0)),
        ],
        out_specs=pl.BlockSpec((sm, n), lambda i: (i, 0)),
        scratch_shapes=[pltpu.VMEM((sm, pw), _F32)],
        compiler_params=pltpu.CompilerParams(
            dimension_semantics=("parallel",),
            vmem_limit_bytes=_VMEM_LIMIT),
    )(bounds, keys2d)


def _proj_kernel(x_ref, wl_ref, wr_ref, c0_ref, rs_ref, xwl_ref, self_ref):
    xb = x_ref[...].astype(_BF16)
    xwl_ref[...] = jnp.dot(xb, wl_ref[...],
                           preferred_element_type=_F32).astype(_BF16)
    c0 = c0_ref[0:1, :]
    c1 = c0_ref[1:2, :]
    self_ref[...] = (jnp.dot(xb, wr_ref[...], preferred_element_type=_F32)
                     + c0 + rs_ref[...] * c1)


def _layer1_kernel(a_ref, xwl_ref, invdeg_ref, self_ref, wl2_ref, wr2_ref,
                   b2_ref, h1w_ref, h1r_ref, acc_ref):
    k = pl.program_id(1)

    @pl.when(k == 0)
    def _():
        acc_ref[...] = jnp.zeros_like(acc_ref)

    acc_ref[...] += jax.lax.dot_general(
        a_ref[...], xwl_ref[...], (((1,), (0,)), ((), ())),
        preferred_element_type=_F32)

    @pl.when(k == pl.num_programs(1) - 1)
    def _():
        h1 = jnp.maximum(acc_ref[...] * invdeg_ref[...] + self_ref[...], 0.0)
        h1b = h1.astype(_BF16)
        h1w_ref[...] = jnp.dot(h1b, wl2_ref[...],
                               preferred_element_type=_F32).astype(_BF16)
        h1r_ref[...] = (jnp.dot(h1b, wr2_ref[...],
                                preferred_element_type=_F32) + b2_ref[...])


def _layer2_kernel(a_ref, h1w_ref, invdeg_ref, self_ref, wo_ref, bo_ref,
                   o_ref, acc_ref):
    k = pl.program_id(1)

    @pl.when(k == 0)
    def _():
        acc_ref[...] = jnp.zeros_like(acc_ref)

    acc_ref[...] += jax.lax.dot_general(
        a_ref[...], h1w_ref[...], (((1,), (0,)), ((), ())),
        preferred_element_type=_F32)

    @pl.when(k == pl.num_programs(1) - 1)
    def _():
        h2 = jnp.maximum(acc_ref[...] * invdeg_ref[...] + self_ref[...], 0.0)
        o_ref[...] = (jnp.sum(h2 * wo_ref[...], axis=-1, keepdims=True)
                      + bo_ref[...])


@functools.partial(jax.jit, static_argnames=("num_nodes",))
def _forward(params, x, edge_index, t, num_nodes):
    in_dim = x.shape[1]
    hidden = params["wt1"].shape[0]
    n = num_nodes
    tile_m, tile_k = min(1024, n), min(4096, n)
    grid = (n // tile_m, n // tile_k)

    te = params["embed"][t[0]][None, :]
    th = te @ params["wt1"] + params["bt1"]
    th = th * jax.nn.sigmoid(th)
    temb = th @ params["wt2"] + params["bt2"]

    wl1x, wl1t = params["wl1"][:in_dim], params["wl1"][in_dim:]
    wr1x, wr1t = params["wr1"][:in_dim], params["wr1"][in_dim:]
    c0 = params["bl1"] + temb @ wr1t
    c1 = temb @ wl1t
    c01 = jnp.concatenate([c0, c1], axis=0)

    src, dst = edge_index[0], edge_index[1]
    a_ind = _build_adjacency(src, dst, n)
    deg = jnp.zeros((n,), _F32).at[dst].add(jnp.ones(src.shape[0], _F32))
    invdeg = (1.0 / jnp.maximum(deg, 1.0))[:, None]
    rowsum = (deg > 0).astype(_F32)[:, None]

    row = lambda r, c: pl.BlockSpec((r, c), lambda i, k: (i, 0))
    panel = lambda r, c: pl.BlockSpec((r, c), lambda i, k: (k, 0))
    const = lambda shape: pl.BlockSpec(shape, lambda i, k: (0, 0))
    a_spec = pl.BlockSpec((tile_m, tile_k), lambda i, k: (i, k))
    cparams = pltpu.CompilerParams(
        dimension_semantics=("parallel", "arbitrary"),
        vmem_limit_bytes=_VMEM_LIMIT)

    proj_m = min(2048, n)
    xwl, self1 = pl.pallas_call(
        _proj_kernel,
        out_shape=(jax.ShapeDtypeStruct((n, hidden), _BF16),
                   jax.ShapeDtypeStruct((n, hidden), _F32)),
        grid=(n // proj_m,),
        in_specs=[
            pl.BlockSpec((proj_m, in_dim), lambda i: (i, 0)),
            pl.BlockSpec((in_dim, hidden), lambda i: (0, 0)),
            pl.BlockSpec((in_dim, hidden), lambda i: (0, 0)),
            pl.BlockSpec((2, hidden), lambda i: (0, 0)),
            pl.BlockSpec((proj_m, 1), lambda i: (i, 0)),
        ],
        out_specs=[pl.BlockSpec((proj_m, hidden), lambda i: (i, 0)),
                   pl.BlockSpec((proj_m, hidden), lambda i: (i, 0))],
        compiler_params=pltpu.CompilerParams(
            dimension_semantics=("parallel",),
            vmem_limit_bytes=_VMEM_LIMIT),
    )(x, wl1x.astype(_BF16), wr1x.astype(_BF16), c01, rowsum)

    wl2_bf = params["wl2"].astype(_BF16)
    wr2_bf = params["wr2"].astype(_BF16)

    h1w, h1r = pl.pallas_call(
        _layer1_kernel,
        out_shape=(jax.ShapeDtypeStruct((n, hidden), _BF16),
                   jax.ShapeDtypeStruct((n, hidden), _F32)),
        grid=grid,
        in_specs=[
            a_spec,
            panel(tile_k, hidden),
            row(tile_m, 1),
            row(tile_m, hidden),
            const((hidden, hidden)),
            const((hidden, hidden)),
            const((1, hidden)),
        ],
        out_specs=[row(tile_m, hidden), row(tile_m, hidden)],
        scratch_shapes=[pltpu.VMEM((tile_m, hidden), _F32)],
        compiler_params=cparams,
    )(a_ind, xwl, invdeg, self1, wl2_bf, wr2_bf, params["bl2"])

    out = pl.pallas_call(
        _layer2_kernel,
        out_shape=jax.ShapeDtypeStruct((n, 1), _F32),
        grid=grid,
        in_specs=[
            a_spec,
            panel(tile_k, hidden),
            row(tile_m, 1),
            row(tile_m, hidden),
            const((1, hidden)),
            const((1, 1)),
        ],
        out_specs=row(tile_m, 1),
        scratch_shapes=[pltpu.VMEM((tile_m, hidden), _F32)],
        compiler_params=cparams,
    )(a_ind, h1w, invdeg, h1r, params["wo"].T, params["bo"])

    return out[:, 0]


def kernel(embed, wt1, bt1, wt2, bt2, wl1, bl1, wr1, wl2, bl2, wr2, wo, bo,
           x, edge_index, t):
    params = {
        "embed": embed, "wt1": wt1, "bt1": bt1, "wt2": wt2, "bt2": bt2,
        "wl1": wl1, "bl1": bl1, "wr1": wr1, "wl2": wl2, "bl2": bl2,
        "wr2": wr2, "wo": wo, "bo": bo,
    }
    return _forward(params, x, edge_index, t, num_nodes=x.shape[0])


# speedup vs baseline: 3.7014x; 1.1124x over previous
import functools

import jax
import jax.numpy as jnp
from jax.experimental import pallas as pl
from jax.experimental.pallas import tpu as pltpu

_F32 = jnp.float32
_BF16 = jnp.bfloat16
_FP8 = jnp.float8_e4m3fn

_VMEM_LIMIT = 50 << 20


def _adj_build_kernel(n_panels, sm, pw, pb_bits,
                      bounds_ref, keys_ref, out_ref, acc_ref):
    i = pl.program_id(0)
    lane = jax.lax.broadcasted_iota(jnp.int32, (1, 128), 1)
    sub_s = jax.lax.broadcasted_iota(jnp.int32, (pw, 128), 0)
    sub_d = jax.lax.broadcasted_iota(jnp.int32, (sm, 128), 0)

    def onehots(row, start, end):
        k = keys_ref[pl.ds(row, 1), :]
        e0 = row << 7
        valid = ((e0 + lane) >= start) & ((e0 + lane) < end)
        sl = jnp.where(valid, k & (pw - 1), pw)
        dl = (k >> pb_bits) & (sm - 1)
        return (sub_s == sl), (sub_d == dl)

    for p in range(n_panels):
        pid = i * n_panels + p
        start = bounds_ref[pid]
        end = bounds_ref[pid + 1]
        base_row = start >> 7

        parts = [onehots(base_row + j, start, end) for j in range(4)]
        s_oh = jnp.concatenate([s for s, _ in parts], axis=1).astype(_FP8)
        d_oh = jnp.concatenate([d for _, d in parts], axis=1).astype(_FP8)
        res = jax.lax.dot_general(d_oh, s_oh, (((1,), (1,)), ((), ())),
                                  preferred_element_type=_F32)
        out_ref[:, p * pw:(p + 1) * pw] = res.astype(out_ref.dtype)

        @pl.when(end > (base_row << 7) + 512)
        def _():
            acc_ref[...] = jnp.zeros_like(acc_ref)
            nch = (end - (base_row << 7) + 255) >> 8

            def body(c, carry):
                row = base_row + 2 * c
                s0, d0 = onehots(row, start, end)
                s1, d1 = onehots(row + 1, start, end)
                s2 = jnp.concatenate([s0, s1], axis=1).astype(_FP8)
                d2 = jnp.concatenate([d0, d1], axis=1).astype(_FP8)
                acc_ref[...] += jax.lax.dot_general(
                    d2, s2, (((1,), (1,)), ((), ())),
                    preferred_element_type=_F32)
                return carry

            jax.lax.fori_loop(2, nch, body, 0)
            total = out_ref[:, p * pw:(p + 1) * pw].astype(_F32) + acc_ref[...]
            out_ref[:, p * pw:(p + 1) * pw] = total.astype(out_ref.dtype)


def _deg_kernel(cps, dst_ref, out_ref):
    s = pl.program_id(0)
    iota = jax.lax.broadcasted_iota(jnp.int32, (128, 128), 0)
    acc = jnp.zeros((128, 128), _F32)
    for c in range(cps):
        rows = [dst_ref[pl.ds(4 * c + r, 1), :] for r in range(4)]
        h_oh = jnp.concatenate([iota == (r >> 7) for r in rows],
                               axis=1).astype(_FP8)
        l_oh = jnp.concatenate([iota == (r & 127) for r in rows],
                               axis=1).astype(_FP8)
        acc = acc + jax.lax.dot_general(h_oh, l_oh, (((1,), (1,)), ((), ())),
                                        preferred_element_type=_F32)

    @pl.when(s == 0)
    def _():
        out_ref[...] = acc

    @pl.when(s > 0)
    def _():
        out_ref[...] += acc


def _degree_histogram(dst, n):
    e = dst.shape[0]
    rows = e // 128
    rps = min(128, rows)
    dst2d = dst.reshape(rows, 128)
    hist = pl.pallas_call(
        functools.partial(_deg_kernel, rps // 4),
        out_shape=jax.ShapeDtypeStruct((128, 128), _F32),
        grid=(rows // rps,),
        in_specs=[pl.BlockSpec((rps, 128), lambda s: (s, 0))],
        out_specs=pl.BlockSpec((128, 128), lambda s: (0, 0)),
        compiler_params=pltpu.CompilerParams(
            dimension_semantics=("arbitrary",),
            vmem_limit_bytes=_VMEM_LIMIT),
    )(dst2d)
    return hist.reshape(-1)[:n]


def _searchsorted_rows(sk, starts):
    e = sk.shape[0]
    rows = e // 128
    sk2d = sk.reshape(rows, 128)
    heads = sk2d[:, 0]
    c = jnp.sum(heads[None, :] < starts[:, None], axis=1)
    row = jnp.maximum(c - 1, 0)
    oh = (row[:, None] == jnp.arange(rows)[None, :]).astype(_F32)
    hi = (sk2d >> 14).astype(_F32)
    lo = (sk2d & 0x3FFF).astype(_F32)
    vals = (jnp.dot(oh, hi).astype(jnp.int32) << 14) | \
        jnp.dot(oh, lo).astype(jnp.int32)
    cnt = jnp.sum(vals < starts[:, None], axis=1)
    return (row * 128 + cnt).astype(jnp.int32)


def _build_adjacency(src, dst, n):
    e = src.shape[0]
    sm, pw = min(256, n), min(1024, n)
    n_strips, n_panels = n // sm, n // pw
    pb_bits = (pw - 1).bit_length()
    db_bits = (sm - 1).bit_length()

    strip = dst // sm
    panel = src // pw
    key = ((((strip * n_panels) + panel) << (db_bits + pb_bits))
           | ((dst & (sm - 1)) << pb_bits) | (src & (pw - 1)))
    sk = jax.lax.sort(key)

    npairs = n_strips * n_panels
    starts = (jnp.arange(npairs + 1, dtype=jnp.int32)
              << (db_bits + pb_bits))
    bounds = jnp.searchsorted(sk, starts, side='left').astype(jnp.int32)

    rows = e // 128
    keys2d = jnp.concatenate(
        [sk, jnp.full((8 * 128,), jnp.iinfo(jnp.int32).max, jnp.int32)]
    ).reshape(rows + 8, 128)

    return pl.pallas_call(
        functools.partial(_adj_build_kernel, n_panels, sm, pw, pb_bits),
        out_shape=jax.ShapeDtypeStruct((n, n), _FP8),
        grid=(n_strips,),
        in_specs=[
            pl.BlockSpec(memory_space=pltpu.SMEM),
            pl.BlockSpec((rows + 8, 128), lambda i: (0, 0)),
        ],
        out_specs=pl.BlockSpec((sm, n), lambda i: (i, 0)),
        scratch_shapes=[pltpu.VMEM((sm, pw), _F32)],
        compiler_params=pltpu.CompilerParams(
            dimension_semantics=("parallel",),
            vmem_limit_bytes=_VMEM_LIMIT),
    )(bounds, keys2d)


def _proj_kernel(x_ref, wl_ref, wr_ref, c0_ref, rs_ref, xwl_ref, self_ref):
    xb = x_ref[...].astype(_BF16)
    xwl_ref[...] = jnp.dot(xb, wl_ref[...],
                           preferred_element_type=_F32).astype(_BF16)
    c0 = c0_ref[0:1, :]
    c1 = c0_ref[1:2, :]
    self_ref[...] = (jnp.dot(xb, wr_ref[...], preferred_element_type=_F32)
                     + c0 + rs_ref[...] * c1)


def _layer1_kernel(a_ref, xwl_ref, invdeg_ref, self_ref, wl2_ref, wr2_ref,
                   b2_ref, h1w_ref, h1r_ref, acc_ref):
    k = pl.program_id(1)

    @pl.when(k == 0)
    def _():
        acc_ref[...] = jnp.zeros_like(acc_ref)

    acc_ref[...] += jax.lax.dot_general(
        a_ref[...], xwl_ref[...], (((1,), (0,)), ((), ())),
        preferred_element_type=_F32)

    @pl.when(k == pl.num_programs(1) - 1)
    def _():
        h1 = jnp.maximum(acc_ref[...] * invdeg_ref[...] + self_ref[...], 0.0)
        h1b = h1.astype(_BF16)
        h1w_ref[...] = jnp.dot(h1b, wl2_ref[...],
                               preferred_element_type=_F32).astype(_BF16)
        h1r_ref[...] = (jnp.dot(h1b, wr2_ref[...],
                                preferred_element_type=_F32) + b2_ref[...])


def _layer2_kernel(a_ref, h1w_ref, invdeg_ref, self_ref, wo_ref, bo_ref,
                   o_ref, acc_ref):
    k = pl.program_id(1)

    @pl.when(k == 0)
    def _():
        acc_ref[...] = jnp.zeros_like(acc_ref)

    acc_ref[...] += jax.lax.dot_general(
        a_ref[...], h1w_ref[...], (((1,), (0,)), ((), ())),
        preferred_element_type=_F32)

    @pl.when(k == pl.num_programs(1) - 1)
    def _():
        h2 = jnp.maximum(acc_ref[...] * invdeg_ref[...] + self_ref[...], 0.0)
        o_ref[...] = (jnp.sum(h2 * wo_ref[...], axis=-1, keepdims=True)
                      + bo_ref[...])


@functools.partial(jax.jit, static_argnames=("num_nodes",))
def _forward(params, x, edge_index, t, num_nodes):
    in_dim = x.shape[1]
    hidden = params["wt1"].shape[0]
    n = num_nodes
    tile_m, tile_k = min(1024, n), min(4096, n)
    grid = (n // tile_m, n // tile_k)

    te = params["embed"][t[0]][None, :]
    th = te @ params["wt1"] + params["bt1"]
    th = th * jax.nn.sigmoid(th)
    temb = th @ params["wt2"] + params["bt2"]

    wl1x, wl1t = params["wl1"][:in_dim], params["wl1"][in_dim:]
    wr1x, wr1t = params["wr1"][:in_dim], params["wr1"][in_dim:]
    c0 = params["bl1"] + temb @ wr1t
    c1 = temb @ wl1t
    c01 = jnp.concatenate([c0, c1], axis=0)

    src, dst = edge_index[0], edge_index[1]
    a_ind = _build_adjacency(src, dst, n)
    deg = _degree_histogram(dst, n)
    invdeg = (1.0 / jnp.maximum(deg, 1.0))[:, None]
    rowsum = (deg > 0).astype(_F32)[:, None]

    row = lambda r, c: pl.BlockSpec((r, c), lambda i, k: (i, 0))
    panel = lambda r, c: pl.BlockSpec((r, c), lambda i, k: (k, 0))
    const = lambda shape: pl.BlockSpec(shape, lambda i, k: (0, 0))
    a_spec = pl.BlockSpec((tile_m, tile_k), lambda i, k: (i, k))
    cparams = pltpu.CompilerParams(
        dimension_semantics=("parallel", "arbitrary"),
        vmem_limit_bytes=_VMEM_LIMIT)

    proj_m = min(2048, n)
    xwl, self1 = pl.pallas_call(
        _proj_kernel,
        out_shape=(jax.ShapeDtypeStruct((n, hidden), _BF16),
                   jax.ShapeDtypeStruct((n, hidden), _F32)),
        grid=(n // proj_m,),
        in_specs=[
            pl.BlockSpec((proj_m, in_dim), lambda i: (i, 0)),
            pl.BlockSpec((in_dim, hidden), lambda i: (0, 0)),
            pl.BlockSpec((in_dim, hidden), lambda i: (0, 0)),
            pl.BlockSpec((2, hidden), lambda i: (0, 0)),
            pl.BlockSpec((proj_m, 1), lambda i: (i, 0)),
        ],
        out_specs=[pl.BlockSpec((proj_m, hidden), lambda i: (i, 0)),
                   pl.BlockSpec((proj_m, hidden), lambda i: (i, 0))],
        compiler_params=pltpu.CompilerParams(
            dimension_semantics=("parallel",),
            vmem_limit_bytes=_VMEM_LIMIT),
    )(x, wl1x.astype(_BF16), wr1x.astype(_BF16), c01, rowsum)

    wl2_bf = params["wl2"].astype(_BF16)
    wr2_bf = params["wr2"].astype(_BF16)

    h1w, h1r = pl.pallas_call(
        _layer1_kernel,
        out_shape=(jax.ShapeDtypeStruct((n, hidden), _BF16),
                   jax.ShapeDtypeStruct((n, hidden), _F32)),
        grid=grid,
        in_specs=[
            a_spec,
            panel(tile_k, hidden),
            row(tile_m, 1),
            row(tile_m, hidden),
            const((hidden, hidden)),
            const((hidden, hidden)),
            const((1, hidden)),
        ],
        out_specs=[row(tile_m, hidden), row(tile_m, hidden)],
        scratch_shapes=[pltpu.VMEM((tile_m, hidden), _F32)],
        compiler_params=cparams,
    )(a_ind, xwl, invdeg, self1, wl2_bf, wr2_bf, params["bl2"])

    out = pl.pallas_call(
        _layer2_kernel,
        out_shape=jax.ShapeDtypeStruct((n, 1), _F32),
        grid=grid,
        in_specs=[
            a_spec,
            panel(tile_k, hidden),
            row(tile_m, 1),
            row(tile_m, hidden),
            const((1, hidden)),
            const((1, 1)),
        ],
        out_specs=row(tile_m, 1),
        scratch_shapes=[pltpu.VMEM((tile_m, hidden), _F32)],
        compiler_params=cparams,
    )(a_ind, h1w, invdeg, h1r, params["wo"].T, params["bo"])

    return out[:, 0]


def kernel(embed, wt1, bt1, wt2, bt2, wl1, bl1, wr1, wl2, bl2, wr2, wo, bo,
           x, edge_index, t):
    params = {
        "embed": embed, "wt1": wt1, "bt1": bt1, "wt2": wt2, "bt2": bt2,
        "wl1": wl1, "bl1": bl1, "wr1": wr1, "wl2": wl2, "bl2": bl2,
        "wr2": wr2, "wo": wo, "bo": bo,
    }
    return _forward(params, x, edge_index, t, num_nodes=x.shape[0])


# speedup vs baseline: 3.7832x; 1.0221x over previous
import functools

import jax
import jax.numpy as jnp
from jax.experimental import pallas as pl
from jax.experimental.pallas import tpu as pltpu

_F32 = jnp.float32
_BF16 = jnp.bfloat16
_FP8 = jnp.float8_e4m3fn

_VMEM_LIMIT = 50 << 20


def _adj_build_kernel(n_panels, sm, pw, pb_bits,
                      bounds_ref, keys_ref, out_ref, acc_ref):
    i = pl.program_id(0)
    lane = jax.lax.broadcasted_iota(jnp.int32, (1, 128), 1)
    sub_s = jax.lax.broadcasted_iota(jnp.int32, (pw, 128), 0)
    sub_d = jax.lax.broadcasted_iota(jnp.int32, (sm, 128), 0)

    def onehots(row, start, end):
        k = keys_ref[pl.ds(row, 1), :]
        e0 = row << 7
        valid = ((e0 + lane) >= start) & ((e0 + lane) < end)
        sl = jnp.where(valid, k & (pw - 1), pw)
        dl = (k >> pb_bits) & (sm - 1)
        return (sub_s == sl), (sub_d == dl)

    for p in range(n_panels):
        pid = i * n_panels + p
        start = bounds_ref[pid]
        end = bounds_ref[pid + 1]
        base_row = start >> 7

        parts = [onehots(base_row + j, start, end) for j in range(3)]
        s_oh = jnp.concatenate([s for s, _ in parts], axis=1).astype(_FP8)
        d_oh = jnp.concatenate([d for _, d in parts], axis=1).astype(_FP8)
        res = jax.lax.dot_general(d_oh, s_oh, (((1,), (1,)), ((), ())),
                                  preferred_element_type=_F32)
        out_ref[:, p * pw:(p + 1) * pw] = res.astype(out_ref.dtype)

        @pl.when(end > (base_row << 7) + 384)
        def _():
            acc_ref[...] = jnp.zeros_like(acc_ref)
            ntail = (end - (base_row << 7) - 384 + 255) >> 8

            def body(c, carry):
                row = base_row + 3 + 2 * c
                s0, d0 = onehots(row, start, end)
                s1, d1 = onehots(row + 1, start, end)
                s2 = jnp.concatenate([s0, s1], axis=1).astype(_FP8)
                d2 = jnp.concatenate([d0, d1], axis=1).astype(_FP8)
                acc_ref[...] += jax.lax.dot_general(
                    d2, s2, (((1,), (1,)), ((), ())),
                    preferred_element_type=_F32)
                return carry

            jax.lax.fori_loop(0, ntail, body, 0)
            total = out_ref[:, p * pw:(p + 1) * pw].astype(_F32) + acc_ref[...]
            out_ref[:, p * pw:(p + 1) * pw] = total.astype(out_ref.dtype)


def _deg_kernel(cps, dst_ref, out_ref):
    s = pl.program_id(0)
    iota = jax.lax.broadcasted_iota(jnp.int32, (128, 128), 0)
    acc = jnp.zeros((128, 128), _F32)
    for c in range(cps):
        rows = [dst_ref[pl.ds(4 * c + r, 1), :] for r in range(4)]
        h_oh = jnp.concatenate([iota == (r >> 7) for r in rows],
                               axis=1).astype(_FP8)
        l_oh = jnp.concatenate([iota == (r & 127) for r in rows],
                               axis=1).astype(_FP8)
        acc = acc + jax.lax.dot_general(h_oh, l_oh, (((1,), (1,)), ((), ())),
                                        preferred_element_type=_F32)

    @pl.when(s == 0)
    def _():
        out_ref[...] = acc

    @pl.when(s > 0)
    def _():
        out_ref[...] += acc


def _degree_histogram(dst, n):
    e = dst.shape[0]
    rows = e // 128
    rps = min(128, rows)
    dst2d = dst.reshape(rows, 128)
    hist = pl.pallas_call(
        functools.partial(_deg_kernel, rps // 4),
        out_shape=jax.ShapeDtypeStruct((128, 128), _F32),
        grid=(rows // rps,),
        in_specs=[pl.BlockSpec((rps, 128), lambda s: (s, 0))],
        out_specs=pl.BlockSpec((128, 128), lambda s: (0, 0)),
        compiler_params=pltpu.CompilerParams(
            dimension_semantics=("arbitrary",),
            vmem_limit_bytes=_VMEM_LIMIT),
    )(dst2d)
    return hist.reshape(-1)[:n]


def _searchsorted_rows(sk, starts):
    e = sk.shape[0]
    rows = e // 128
    sk2d = sk.reshape(rows, 128)
    heads = sk2d[:, 0]
    c = jnp.sum(heads[None, :] < starts[:, None], axis=1)
    row = jnp.maximum(c - 1, 0)
    oh = (row[:, None] == jnp.arange(rows)[None, :]).astype(_F32)
    hi = (sk2d >> 14).astype(_F32)
    lo = (sk2d & 0x3FFF).astype(_F32)
    vals = (jnp.dot(oh, hi).astype(jnp.int32) << 14) | \
        jnp.dot(oh, lo).astype(jnp.int32)
    cnt = jnp.sum(vals < starts[:, None], axis=1)
    return (row * 128 + cnt).astype(jnp.int32)


def _build_adjacency(src, dst, n):
    e = src.shape[0]
    sm, pw = min(256, n), min(1024, n)
    n_strips, n_panels = n // sm, n // pw
    pb_bits = (pw - 1).bit_length()
    db_bits = (sm - 1).bit_length()

    strip = dst // sm
    panel = src // pw
    key = ((((strip * n_panels) + panel) << (db_bits + pb_bits))
           | ((dst & (sm - 1)) << pb_bits) | (src & (pw - 1)))
    sk = jax.lax.sort(key)

    npairs = n_strips * n_panels
    starts = (jnp.arange(npairs + 1, dtype=jnp.int32)
              << (db_bits + pb_bits))
    bounds = jnp.searchsorted(sk, starts, side='left').astype(jnp.int32)

    rows = e // 128
    keys2d = jnp.concatenate(
        [sk, jnp.full((8 * 128,), jnp.iinfo(jnp.int32).max, jnp.int32)]
    ).reshape(rows + 8, 128)

    return pl.pallas_call(
        functools.partial(_adj_build_kernel, n_panels, sm, pw, pb_bits),
        out_shape=jax.ShapeDtypeStruct((n, n), _FP8),
        grid=(n_strips,),
        in_specs=[
            pl.BlockSpec(memory_space=pltpu.SMEM),
            pl.BlockSpec((rows + 8, 128), lambda i: (0, 0)),
        ],
        out_specs=pl.BlockSpec((sm, n), lambda i: (i, 0)),
        scratch_shapes=[pltpu.VMEM((sm, pw), _F32)],
        compiler_params=pltpu.CompilerParams(
            dimension_semantics=("parallel",),
            vmem_limit_bytes=_VMEM_LIMIT),
    )(bounds, keys2d)


def _proj_kernel(x_ref, wl_ref, wr_ref, c0_ref, rs_ref, xwl_ref, self_ref):
    xb = x_ref[...].astype(_BF16)
    xwl_ref[...] = jnp.dot(xb, wl_ref[...],
                           preferred_element_type=_F32).astype(_BF16)
    c0 = c0_ref[0:1, :]
    c1 = c0_ref[1:2, :]
    self_ref[...] = (jnp.dot(xb, wr_ref[...], preferred_element_type=_F32)
                     + c0 + rs_ref[...] * c1)


def _layer1_kernel(a_ref, xwl_ref, invdeg_ref, self_ref, wl2_ref, wr2_ref,
                   b2_ref, h1w_ref, h1r_ref, acc_ref):
    k = pl.program_id(1)

    @pl.when(k == 0)
    def _():
        acc_ref[...] = jnp.zeros_like(acc_ref)

    acc_ref[...] += jax.lax.dot_general(
        a_ref[...], xwl_ref[...], (((1,), (0,)), ((), ())),
        preferred_element_type=_F32)

    @pl.when(k == pl.num_programs(1) - 1)
    def _():
        h1 = jnp.maximum(acc_ref[...] * invdeg_ref[...] + self_ref[...], 0.0)
        h1b = h1.astype(_BF16)
        h1w_ref[...] = jnp.dot(h1b, wl2_ref[...],
                               preferred_element_type=_F32).astype(_BF16)
        h1r_ref[...] = (jnp.dot(h1b, wr2_ref[...],
                                preferred_element_type=_F32) + b2_ref[...])


def _layer2_kernel(a_ref, h1w_ref, invdeg_ref, self_ref, wo_ref, bo_ref,
                   o_ref, acc_ref):
    k = pl.program_id(1)

    @pl.when(k == 0)
    def _():
        acc_ref[...] = jnp.zeros_like(acc_ref)

    acc_ref[...] += jax.lax.dot_general(
        a_ref[...], h1w_ref[...], (((1,), (0,)), ((), ())),
        preferred_element_type=_F32)

    @pl.when(k == pl.num_programs(1) - 1)
    def _():
        h2 = jnp.maximum(acc_ref[...] * invdeg_ref[...] + self_ref[...], 0.0)
        o_ref[...] = (jnp.sum(h2 * wo_ref[...], axis=-1, keepdims=True)
                      + bo_ref[...])


@functools.partial(jax.jit, static_argnames=("num_nodes",))
def _forward(params, x, edge_index, t, num_nodes):
    in_dim = x.shape[1]
    hidden = params["wt1"].shape[0]
    n = num_nodes
    tile_m, tile_k = min(1024, n), min(4096, n)
    grid = (n // tile_m, n // tile_k)

    te = params["embed"][t[0]][None, :]
    th = te @ params["wt1"] + params["bt1"]
    th = th * jax.nn.sigmoid(th)
    temb = th @ params["wt2"] + params["bt2"]

    wl1x, wl1t = params["wl1"][:in_dim], params["wl1"][in_dim:]
    wr1x, wr1t = params["wr1"][:in_dim], params["wr1"][in_dim:]
    c0 = params["bl1"] + temb @ wr1t
    c1 = temb @ wl1t
    c01 = jnp.concatenate([c0, c1], axis=0)

    src, dst = edge_index[0], edge_index[1]
    a_ind = _build_adjacency(src, dst, n)
    deg = _degree_histogram(dst, n)
    invdeg = (1.0 / jnp.maximum(deg, 1.0))[:, None]
    rowsum = (deg > 0).astype(_F32)[:, None]

    row = lambda r, c: pl.BlockSpec((r, c), lambda i, k: (i, 0))
    panel = lambda r, c: pl.BlockSpec((r, c), lambda i, k: (k, 0))
    const = lambda shape: pl.BlockSpec(shape, lambda i, k: (0, 0))
    a_spec = pl.BlockSpec((tile_m, tile_k), lambda i, k: (i, k))
    cparams = pltpu.CompilerParams(
        dimension_semantics=("parallel", "arbitrary"),
        vmem_limit_bytes=_VMEM_LIMIT)

    proj_m = min(2048, n)
    xwl, self1 = pl.pallas_call(
        _proj_kernel,
        out_shape=(jax.ShapeDtypeStruct((n, hidden), _BF16),
                   jax.ShapeDtypeStruct((n, hidden), _F32)),
        grid=(n // proj_m,),
        in_specs=[
            pl.BlockSpec((proj_m, in_dim), lambda i: (i, 0)),
            pl.BlockSpec((in_dim, hidden), lambda i: (0, 0)),
            pl.BlockSpec((in_dim, hidden), lambda i: (0, 0)),
            pl.BlockSpec((2, hidden), lambda i: (0, 0)),
            pl.BlockSpec((proj_m, 1), lambda i: (i, 0)),
        ],
        out_specs=[pl.BlockSpec((proj_m, hidden), lambda i: (i, 0)),
                   pl.BlockSpec((proj_m, hidden), lambda i: (i, 0))],
        compiler_params=pltpu.CompilerParams(
            dimension_semantics=("parallel",),
            vmem_limit_bytes=_VMEM_LIMIT),
    )(x, wl1x.astype(_BF16), wr1x.astype(_BF16), c01, rowsum)

    wl2_bf = params["wl2"].astype(_BF16)
    wr2_bf = params["wr2"].astype(_BF16)

    h1w, h1r = pl.pallas_call(
        _layer1_kernel,
        out_shape=(jax.ShapeDtypeStruct((n, hidden), _BF16),
                   jax.ShapeDtypeStruct((n, hidden), _F32)),
        grid=grid,
        in_specs=[
            a_spec,
            panel(tile_k, hidden),
            row(tile_m, 1),
            row(tile_m, hidden),
            const((hidden, hidden)),
            const((hidden, hidden)),
            const((1, hidden)),
        ],
        out_specs=[row(tile_m, hidden), row(tile_m, hidden)],
        scratch_shapes=[pltpu.VMEM((tile_m, hidden), _F32)],
        compiler_params=cparams,
    )(a_ind, xwl, invdeg, self1, wl2_bf, wr2_bf, params["bl2"])

    out = pl.pallas_call(
        _layer2_kernel,
        out_shape=jax.ShapeDtypeStruct((n, 1), _F32),
        grid=grid,
        in_specs=[
            a_spec,
            panel(tile_k, hidden),
            row(tile_m, 1),
            row(tile_m, hidden),
            const((1, hidden)),
            const((1, 1)),
        ],
        out_specs=row(tile_m, 1),
        scratch_shapes=[pltpu.VMEM((tile_m, hidden), _F32)],
        compiler_params=cparams,
    )(a_ind, h1w, invdeg, h1r, params["wo"].T, params["bo"])

    return out[:, 0]


def kernel(embed, wt1, bt1, wt2, bt2, wl1, bl1, wr1, wl2, bl2, wr2, wo, bo,
           x, edge_index, t):
    params = {
        "embed": embed, "wt1": wt1, "bt1": bt1, "wt2": wt2, "bt2": bt2,
        "wl1": wl1, "bl1": bl1, "wr1": wr1, "wl2": wl2, "bl2": bl2,
        "wr2": wr2, "wo": wo, "bo": bo,
    }
    return _forward(params, x, edge_index, t, num_nodes=x.shape[0])


# speedup vs baseline: 3.8991x; 1.0306x over previous
import functools

import jax
import jax.numpy as jnp
from jax.experimental import pallas as pl
from jax.experimental.pallas import tpu as pltpu

_F32 = jnp.float32
_BF16 = jnp.bfloat16
_FP8 = jnp.float8_e4m3fn

_VMEM_LIMIT = 50 << 20


def _adj_build_kernel(n_panels, sm, pw, pb_bits,
                      bounds_ref, keys_ref, out_ref, acc_ref):
    i = pl.program_id(0)
    lane = jax.lax.broadcasted_iota(jnp.int32, (1, 128), 1)
    sub_s = jax.lax.broadcasted_iota(jnp.int32, (pw, 128), 0)
    sub_d = jax.lax.broadcasted_iota(jnp.int32, (sm, 128), 0)

    def onehots(row, start, end):
        k = keys_ref[pl.ds(row, 1), :]
        e0 = row << 7
        valid = ((e0 + lane) >= start) & ((e0 + lane) < end)
        sl = jnp.where(valid, k & (pw - 1), pw)
        dl = (k >> pb_bits) & (sm - 1)
        return (sub_s == sl), (sub_d == dl)

    for p in range(n_panels):
        pid = i * n_panels + p
        start = bounds_ref[pid]
        end = bounds_ref[pid + 1]
        base_row = start >> 7

        parts = [onehots(base_row + j, start, end) for j in range(3)]
        s_oh = jnp.concatenate([s for s, _ in parts], axis=1).astype(_FP8)
        d_oh = jnp.concatenate([d for _, d in parts], axis=1).astype(_FP8)
        res = jax.lax.dot_general(d_oh, s_oh, (((1,), (1,)), ((), ())),
                                  preferred_element_type=_F32)
        out_ref[:, p * pw:(p + 1) * pw] = res.astype(out_ref.dtype)

        @pl.when(end > (base_row << 7) + 384)
        def _():
            acc_ref[...] = jnp.zeros_like(acc_ref)
            ntail = (end - (base_row << 7) - 384 + 255) >> 8

            def body(c, carry):
                row = base_row + 3 + 2 * c
                s0, d0 = onehots(row, start, end)
                s1, d1 = onehots(row + 1, start, end)
                s2 = jnp.concatenate([s0, s1], axis=1).astype(_FP8)
                d2 = jnp.concatenate([d0, d1], axis=1).astype(_FP8)
                acc_ref[...] += jax.lax.dot_general(
                    d2, s2, (((1,), (1,)), ((), ())),
                    preferred_element_type=_F32)
                return carry

            jax.lax.fori_loop(0, ntail, body, 0)
            total = out_ref[:, p * pw:(p + 1) * pw].astype(_F32) + acc_ref[...]
            out_ref[:, p * pw:(p + 1) * pw] = total.astype(out_ref.dtype)


def _deg_kernel(cps, dst_ref, out_ref):
    s = pl.program_id(0)
    iota = jax.lax.broadcasted_iota(jnp.int32, (128, 128), 0)
    acc = jnp.zeros((128, 128), _F32)
    for c in range(cps):
        rows = [dst_ref[pl.ds(4 * c + r, 1), :] for r in range(4)]
        h_oh = jnp.concatenate([iota == (r >> 7) for r in rows],
                               axis=1).astype(_FP8)
        l_oh = jnp.concatenate([iota == (r & 127) for r in rows],
                               axis=1).astype(_FP8)
        acc = acc + jax.lax.dot_general(h_oh, l_oh, (((1,), (1,)), ((), ())),
                                        preferred_element_type=_F32)

    @pl.when(s == 0)
    def _():
        out_ref[...] = acc

    @pl.when(s > 0)
    def _():
        out_ref[...] += acc


def _degree_histogram(dst, n):
    e = dst.shape[0]
    rows = e // 128
    rps = min(128, rows)
    dst2d = dst.reshape(rows, 128)
    hist = pl.pallas_call(
        functools.partial(_deg_kernel, rps // 4),
        out_shape=jax.ShapeDtypeStruct((128, 128), _F32),
        grid=(rows // rps,),
        in_specs=[pl.BlockSpec((rps, 128), lambda s: (s, 0))],
        out_specs=pl.BlockSpec((128, 128), lambda s: (0, 0)),
        compiler_params=pltpu.CompilerParams(
            dimension_semantics=("arbitrary",),
            vmem_limit_bytes=_VMEM_LIMIT),
    )(dst2d)
    return hist.reshape(-1)[:n]


def _searchsorted_rows(sk, starts):
    e = sk.shape[0]
    rows = e // 128
    sk2d = sk.reshape(rows, 128)
    heads = sk2d[:, 0]
    c = jnp.sum(heads[None, :] < starts[:, None], axis=1)
    row = jnp.maximum(c - 1, 0)
    oh = (row[:, None] == jnp.arange(rows)[None, :]).astype(_F32)
    hi = (sk2d >> 14).astype(_F32)
    lo = (sk2d & 0x3FFF).astype(_F32)
    vals = (jnp.dot(oh, hi).astype(jnp.int32) << 14) | \
        jnp.dot(oh, lo).astype(jnp.int32)
    cnt = jnp.sum(vals < starts[:, None], axis=1)
    return (row * 128 + cnt).astype(jnp.int32)


def _build_adjacency(src, dst, n):
    e = src.shape[0]
    sm, pw = min(256, n), min(1024, n)
    n_strips, n_panels = n // sm, n // pw
    pb_bits = (pw - 1).bit_length()
    db_bits = (sm - 1).bit_length()

    strip = dst // sm
    panel = src // pw
    key = ((((strip * n_panels) + panel) << (db_bits + pb_bits))
           | ((dst & (sm - 1)) << pb_bits) | (src & (pw - 1)))
    sk = jax.lax.sort(key)

    npairs = n_strips * n_panels
    starts = (jnp.arange(npairs + 1, dtype=jnp.int32)
              << (db_bits + pb_bits))
    bounds = jnp.searchsorted(sk, starts, side='left').astype(jnp.int32)

    rows = e // 128
    keys2d = jnp.concatenate(
        [sk, jnp.full((8 * 128,), jnp.iinfo(jnp.int32).max, jnp.int32)]
    ).reshape(rows + 8, 128)

    return pl.pallas_call(
        functools.partial(_adj_build_kernel, n_panels, sm, pw, pb_bits),
        out_shape=jax.ShapeDtypeStruct((n, n), _FP8),
        grid=(n_strips,),
        in_specs=[
            pl.BlockSpec(memory_space=pltpu.SMEM),
            pl.BlockSpec((rows + 8, 128), lambda i: (0, 0)),
        ],
        out_specs=pl.BlockSpec((sm, n), lambda i: (i, 0)),
        scratch_shapes=[pltpu.VMEM((sm, pw), _F32)],
        compiler_params=pltpu.CompilerParams(
            dimension_semantics=("parallel",),
            vmem_limit_bytes=_VMEM_LIMIT),
    )(bounds, keys2d)


def _proj_kernel(x_ref, wl_ref, wr_ref, c0_ref, rs_ref, xwl_ref, self_ref):
    xb = x_ref[...].astype(_BF16)
    xwl_ref[...] = jnp.dot(xb, wl_ref[...],
                           preferred_element_type=_F32).astype(_BF16)
    c0 = c0_ref[0:1, :]
    c1 = c0_ref[1:2, :]
    self_ref[...] = (jnp.dot(xb, wr_ref[...], preferred_element_type=_F32)
                     + c0 + rs_ref[...] * c1)


def _layer1_kernel(a_ref, xwl_ref, invdeg_ref, self_ref, wl2_ref, wr2_ref,
                   b2_ref, h1w_ref, h1r_ref, acc_ref):
    k = pl.program_id(1)

    @pl.when(k == 0)
    def _():
        acc_ref[...] = jnp.zeros_like(acc_ref)

    acc_ref[...] += jax.lax.dot_general(
        a_ref[...], xwl_ref[...], (((1,), (0,)), ((), ())),
        preferred_element_type=_F32)

    @pl.when(k == pl.num_programs(1) - 1)
    def _():
        h1 = jnp.maximum(acc_ref[...] * invdeg_ref[...] + self_ref[...], 0.0)
        h1b = h1.astype(_BF16)
        h1w_ref[...] = jnp.dot(h1b, wl2_ref[...],
                               preferred_element_type=_F32).astype(_BF16)
        h1r_ref[...] = (jnp.dot(h1b, wr2_ref[...],
                                preferred_element_type=_F32) + b2_ref[...])


def _layer2_kernel(a_ref, h1w_ref, invdeg_ref, self_ref, wo_ref, bo_ref,
                   o_ref, acc_ref):
    k = pl.program_id(1)

    @pl.when(k == 0)
    def _():
        acc_ref[...] = jnp.zeros_like(acc_ref)

    acc_ref[...] += jax.lax.dot_general(
        a_ref[...], h1w_ref[...], (((1,), (0,)), ((), ())),
        preferred_element_type=_F32)

    @pl.when(k == pl.num_programs(1) - 1)
    def _():
        h2 = jnp.maximum(acc_ref[...] * invdeg_ref[...] + self_ref[...], 0.0)
        o_ref[...] = (jnp.sum(h2 * wo_ref[...], axis=-1, keepdims=True)
                      + bo_ref[...])


@functools.partial(jax.jit, static_argnames=("num_nodes",))
def _forward(params, x, edge_index, t, num_nodes):
    in_dim = x.shape[1]
    hidden = params["wt1"].shape[0]
    n = num_nodes
    tile_m, tile_k = min(1024, n), min(8192, n)
    grid = (n // tile_m, n // tile_k)

    te = params["embed"][t[0]][None, :]
    th = te @ params["wt1"] + params["bt1"]
    th = th * jax.nn.sigmoid(th)
    temb = th @ params["wt2"] + params["bt2"]

    wl1x, wl1t = params["wl1"][:in_dim], params["wl1"][in_dim:]
    wr1x, wr1t = params["wr1"][:in_dim], params["wr1"][in_dim:]
    c0 = params["bl1"] + temb @ wr1t
    c1 = temb @ wl1t
    c01 = jnp.concatenate([c0, c1], axis=0)

    src, dst = edge_index[0], edge_index[1]
    a_ind = _build_adjacency(src, dst, n)
    deg = _degree_histogram(dst, n)
    invdeg = (1.0 / jnp.maximum(deg, 1.0))[:, None]
    rowsum = (deg > 0).astype(_F32)[:, None]

    row = lambda r, c: pl.BlockSpec((r, c), lambda i, k: (i, 0))
    panel = lambda r, c: pl.BlockSpec((r, c), lambda i, k: (k, 0))
    const = lambda shape: pl.BlockSpec(shape, lambda i, k: (0, 0))
    a_spec = pl.BlockSpec((tile_m, tile_k), lambda i, k: (i, k))
    cparams = pltpu.CompilerParams(
        dimension_semantics=("parallel", "arbitrary"),
        vmem_limit_bytes=_VMEM_LIMIT)

    proj_m = min(2048, n)
    xwl, self1 = pl.pallas_call(
        _proj_kernel,
        out_shape=(jax.ShapeDtypeStruct((n, hidden), _BF16),
                   jax.ShapeDtypeStruct((n, hidden), _F32)),
        grid=(n // proj_m,),
        in_specs=[
            pl.BlockSpec((proj_m, in_dim), lambda i: (i, 0)),
            pl.BlockSpec((in_dim, hidden), lambda i: (0, 0)),
            pl.BlockSpec((in_dim, hidden), lambda i: (0, 0)),
            pl.BlockSpec((2, hidden), lambda i: (0, 0)),
            pl.BlockSpec((proj_m, 1), lambda i: (i, 0)),
        ],
        out_specs=[pl.BlockSpec((proj_m, hidden), lambda i: (i, 0)),
                   pl.BlockSpec((proj_m, hidden), lambda i: (i, 0))],
        compiler_params=pltpu.CompilerParams(
            dimension_semantics=("parallel",),
            vmem_limit_bytes=_VMEM_LIMIT),
    )(x, wl1x.astype(_BF16), wr1x.astype(_BF16), c01, rowsum)

    wl2_bf = params["wl2"].astype(_BF16)
    wr2_bf = params["wr2"].astype(_BF16)

    h1w, h1r = pl.pallas_call(
        _layer1_kernel,
        out_shape=(jax.ShapeDtypeStruct((n, hidden), _BF16),
                   jax.ShapeDtypeStruct((n, hidden), _F32)),
        grid=grid,
        in_specs=[
            a_spec,
            panel(tile_k, hidden),
            row(tile_m, 1),
            row(tile_m, hidden),
            const((hidden, hidden)),
            const((hidden, hidden)),
            const((1, hidden)),
        ],
        out_specs=[row(tile_m, hidden), row(tile_m, hidden)],
        scratch_shapes=[pltpu.VMEM((tile_m, hidden), _F32)],
        compiler_params=cparams,
    )(a_ind, xwl, invdeg, self1, wl2_bf, wr2_bf, params["bl2"])

    out = pl.pallas_call(
        _layer2_kernel,
        out_shape=jax.ShapeDtypeStruct((n, 1), _F32),
        grid=grid,
        in_specs=[
            a_spec,
            panel(tile_k, hidden),
            row(tile_m, 1),
            row(tile_m, hidden),
            const((1, hidden)),
            const((1, 1)),
        ],
        out_specs=row(tile_m, 1),
        scratch_shapes=[pltpu.VMEM((tile_m, hidden), _F32)],
        compiler_params=cparams,
    )(a_ind, h1w, invdeg, h1r, params["wo"].T, params["bo"])

    return out[:, 0]


def kernel(embed, wt1, bt1, wt2, bt2, wl1, bl1, wr1, wl2, bl2, wr2, wo, bo,
           x, edge_index, t):
    params = {
        "embed": embed, "wt1": wt1, "bt1": bt1, "wt2": wt2, "bt2": bt2,
        "wl1": wl1, "bl1": bl1, "wr1": wr1, "wl2": wl2, "bl2": bl2,
        "wr2": wr2, "wo": wo, "bo": bo,
    }
    return _forward(params, x, edge_index, t, num_nodes=x.shape[0])


# speedup vs baseline: 3.9220x; 1.0059x over previous
import functools

import jax
import jax.numpy as jnp
from jax.experimental import pallas as pl
from jax.experimental.pallas import tpu as pltpu

_F32 = jnp.float32
_BF16 = jnp.bfloat16
_FP8 = jnp.float8_e4m3fn

_VMEM_LIMIT = 50 << 20


def _adj_build_kernel(n_panels, sm, pw, pb_bits,
                      bounds_ref, keys_ref, out_ref, acc_ref):
    i = pl.program_id(0)
    lane = jax.lax.broadcasted_iota(jnp.int32, (1, 128), 1)
    sub_s = jax.lax.broadcasted_iota(jnp.int32, (pw, 128), 0)
    sub_d = jax.lax.broadcasted_iota(jnp.int32, (sm, 128), 0)

    def onehots(row, start, end):
        k = keys_ref[pl.ds(row, 1), :]
        e0 = row << 7
        valid = ((e0 + lane) >= start) & ((e0 + lane) < end)
        sl = jnp.where(valid, k & (pw - 1), pw)
        dl = (k >> pb_bits) & (sm - 1)
        return (sub_s == sl), (sub_d == dl)

    for p in range(n_panels):
        pid = i * n_panels + p
        start = bounds_ref[pid]
        end = bounds_ref[pid + 1]
        base_row = start >> 7

        parts = [onehots(base_row + j, start, end) for j in range(3)]
        s_oh = jnp.concatenate([s for s, _ in parts], axis=1).astype(_FP8)
        d_oh = jnp.concatenate([d for _, d in parts], axis=1).astype(_FP8)
        res = jax.lax.dot_general(d_oh, s_oh, (((1,), (1,)), ((), ())),
                                  preferred_element_type=_F32)
        out_ref[:, p * pw:(p + 1) * pw] = res.astype(out_ref.dtype)

        @pl.when(end > (base_row << 7) + 384)
        def _():
            acc_ref[...] = jnp.zeros_like(acc_ref)
            ntail = (end - (base_row << 7) - 384 + 255) >> 8

            def body(c, carry):
                row = base_row + 3 + 2 * c
                s0, d0 = onehots(row, start, end)
                s1, d1 = onehots(row + 1, start, end)
                s2 = jnp.concatenate([s0, s1], axis=1).astype(_FP8)
                d2 = jnp.concatenate([d0, d1], axis=1).astype(_FP8)
                acc_ref[...] += jax.lax.dot_general(
                    d2, s2, (((1,), (1,)), ((), ())),
                    preferred_element_type=_F32)
                return carry

            jax.lax.fori_loop(0, ntail, body, 0)
            total = out_ref[:, p * pw:(p + 1) * pw].astype(_F32) + acc_ref[...]
            out_ref[:, p * pw:(p + 1) * pw] = total.astype(out_ref.dtype)


def _deg_kernel(cps, dst_ref, out_ref):
    s = pl.program_id(0)
    iota = jax.lax.broadcasted_iota(jnp.int32, (128, 128), 0)
    acc = jnp.zeros((128, 128), _F32)
    for c in range(cps):
        rows = [dst_ref[pl.ds(4 * c + r, 1), :] for r in range(4)]
        h_oh = jnp.concatenate([iota == (r >> 7) for r in rows],
                               axis=1).astype(_FP8)
        l_oh = jnp.concatenate([iota == (r & 127) for r in rows],
                               axis=1).astype(_FP8)
        acc = acc + jax.lax.dot_general(h_oh, l_oh, (((1,), (1,)), ((), ())),
                                        preferred_element_type=_F32)

    @pl.when(s == 0)
    def _():
        out_ref[...] = acc

    @pl.when(s > 0)
    def _():
        out_ref[...] += acc


def _degree_histogram(dst, n):
    e = dst.shape[0]
    rows = e // 128
    rps = min(128, rows)
    dst2d = dst.reshape(rows, 128)
    hist = pl.pallas_call(
        functools.partial(_deg_kernel, rps // 4),
        out_shape=jax.ShapeDtypeStruct((128, 128), _F32),
        grid=(rows // rps,),
        in_specs=[pl.BlockSpec((rps, 128), lambda s: (s, 0))],
        out_specs=pl.BlockSpec((128, 128), lambda s: (0, 0)),
        compiler_params=pltpu.CompilerParams(
            dimension_semantics=("arbitrary",),
            vmem_limit_bytes=_VMEM_LIMIT),
    )(dst2d)
    return hist.reshape(-1)[:n]


def _build_adjacency(src, dst, n):
    e = src.shape[0]
    sm, pw = min(256, n), min(1024, n)
    n_strips, n_panels = n // sm, n // pw
    pb_bits = (pw - 1).bit_length()
    db_bits = (sm - 1).bit_length()

    strip = dst // sm
    panel = src // pw
    key = ((((strip * n_panels) + panel) << (db_bits + pb_bits))
           | ((dst & (sm - 1)) << pb_bits) | (src & (pw - 1)))
    sk = jax.lax.sort(key)

    npairs = n_strips * n_panels
    starts = (jnp.arange(npairs + 1, dtype=jnp.int32)
              << (db_bits + pb_bits))
    bounds = jnp.searchsorted(sk, starts, side='left').astype(jnp.int32)

    rows = e // 128
    keys2d = jnp.concatenate(
        [sk, jnp.full((8 * 128,), jnp.iinfo(jnp.int32).max, jnp.int32)]
    ).reshape(rows + 8, 128)

    return pl.pallas_call(
        functools.partial(_adj_build_kernel, n_panels, sm, pw, pb_bits),
        out_shape=jax.ShapeDtypeStruct((n, n), _FP8),
        grid=(n_strips,),
        in_specs=[
            pl.BlockSpec(memory_space=pltpu.SMEM),
            pl.BlockSpec((rows + 8, 128), lambda i: (0, 0)),
        ],
        out_specs=pl.BlockSpec((sm, n), lambda i: (i, 0)),
        scratch_shapes=[pltpu.VMEM((sm, pw), _F32)],
        compiler_params=pltpu.CompilerParams(
            dimension_semantics=("parallel",),
            vmem_limit_bytes=_VMEM_LIMIT),
    )(bounds, keys2d)


def _proj_kernel(x_ref, wl_ref, wr_ref, c0_ref, rs_ref, xwl_ref, self_ref):
    xb = x_ref[...].astype(_BF16)
    xwl_ref[...] = jnp.dot(xb, wl_ref[...],
                           preferred_element_type=_F32).astype(_BF16)
    c0 = c0_ref[0:1, :]
    c1 = c0_ref[1:2, :]
    self_ref[...] = (jnp.dot(xb, wr_ref[...], preferred_element_type=_F32)
                     + c0 + rs_ref[...] * c1)


def _layer1_kernel(a_ref, xwl_ref, invdeg_ref, self_ref, wl2_ref, wr2_ref,
                   b2_ref, h1w_ref, h1r_ref, acc_ref):
    k = pl.program_id(1)

    @pl.when(k == 0)
    def _():
        acc_ref[...] = jnp.zeros_like(acc_ref)

    acc_ref[...] += jax.lax.dot_general(
        a_ref[...], xwl_ref[...], (((1,), (0,)), ((), ())),
        preferred_element_type=_F32)

    @pl.when(k == pl.num_programs(1) - 1)
    def _():
        h1 = jnp.maximum(acc_ref[...] * invdeg_ref[...] + self_ref[...], 0.0)
        h1b = h1.astype(_BF16)
        h1w_ref[...] = jnp.dot(h1b, wl2_ref[...],
                               preferred_element_type=_F32).astype(_BF16)
        h1r_ref[...] = (jnp.dot(h1b, wr2_ref[...],
                                preferred_element_type=_F32) + b2_ref[...])


def _layer2_kernel(a_ref, h1w_ref, invdeg_ref, self_ref, wo_ref, bo_ref,
                   o_ref, acc_ref):
    k = pl.program_id(1)

    @pl.when(k == 0)
    def _():
        acc_ref[...] = jnp.zeros_like(acc_ref)

    acc_ref[...] += jax.lax.dot_general(
        a_ref[...], h1w_ref[...], (((1,), (0,)), ((), ())),
        preferred_element_type=_F32)

    @pl.when(k == pl.num_programs(1) - 1)
    def _():
        h2 = jnp.maximum(acc_ref[...] * invdeg_ref[...] + self_ref[...], 0.0)
        o_ref[...] = (jnp.sum(h2 * wo_ref[...], axis=-1, keepdims=True)
                      + bo_ref[...])


@functools.partial(jax.jit, static_argnames=("num_nodes",))
def _forward(params, x, edge_index, t, num_nodes):
    in_dim = x.shape[1]
    hidden = params["wt1"].shape[0]
    n = num_nodes
    tile_m, tile_k = min(1024, n), min(16384, n)
    grid = (n // tile_m, n // tile_k)

    te = params["embed"][t[0]][None, :]
    th = te @ params["wt1"] + params["bt1"]
    th = th * jax.nn.sigmoid(th)
    temb = th @ params["wt2"] + params["bt2"]

    wl1x, wl1t = params["wl1"][:in_dim], params["wl1"][in_dim:]
    wr1x, wr1t = params["wr1"][:in_dim], params["wr1"][in_dim:]
    c0 = params["bl1"] + temb @ wr1t
    c1 = temb @ wl1t
    c01 = jnp.concatenate([c0, c1], axis=0)

    src, dst = edge_index[0], edge_index[1]
    a_ind = _build_adjacency(src, dst, n)
    deg = _degree_histogram(dst, n)
    invdeg = (1.0 / jnp.maximum(deg, 1.0))[:, None]
    rowsum = (deg > 0).astype(_F32)[:, None]

    row = lambda r, c: pl.BlockSpec((r, c), lambda i, k: (i, 0))
    panel = lambda r, c: pl.BlockSpec((r, c), lambda i, k: (k, 0))
    const = lambda shape: pl.BlockSpec(shape, lambda i, k: (0, 0))
    a_spec = pl.BlockSpec((tile_m, tile_k), lambda i, k: (i, k))
    cparams = pltpu.CompilerParams(
        dimension_semantics=("parallel", "arbitrary"),
        vmem_limit_bytes=_VMEM_LIMIT)

    proj_m = min(2048, n)
    xwl, self1 = pl.pallas_call(
        _proj_kernel,
        out_shape=(jax.ShapeDtypeStruct((n, hidden), _BF16),
                   jax.ShapeDtypeStruct((n, hidden), _F32)),
        grid=(n // proj_m,),
        in_specs=[
            pl.BlockSpec((proj_m, in_dim), lambda i: (i, 0)),
            pl.BlockSpec((in_dim, hidden), lambda i: (0, 0)),
            pl.BlockSpec((in_dim, hidden), lambda i: (0, 0)),
            pl.BlockSpec((2, hidden), lambda i: (0, 0)),
            pl.BlockSpec((proj_m, 1), lambda i: (i, 0)),
        ],
        out_specs=[pl.BlockSpec((proj_m, hidden), lambda i: (i, 0)),
                   pl.BlockSpec((proj_m, hidden), lambda i: (i, 0))],
        compiler_params=pltpu.CompilerParams(
            dimension_semantics=("parallel",),
            vmem_limit_bytes=_VMEM_LIMIT),
    )(x, wl1x.astype(_BF16), wr1x.astype(_BF16), c01, rowsum)

    wl2_bf = params["wl2"].astype(_BF16)
    wr2_bf = params["wr2"].astype(_BF16)

    h1w, h1r = pl.pallas_call(
        _layer1_kernel,
        out_shape=(jax.ShapeDtypeStruct((n, hidden), _BF16),
                   jax.ShapeDtypeStruct((n, hidden), _F32)),
        grid=grid,
        in_specs=[
            a_spec,
            panel(tile_k, hidden),
            row(tile_m, 1),
            row(tile_m, hidden),
            const((hidden, hidden)),
            const((hidden, hidden)),
            const((1, hidden)),
        ],
        out_specs=[row(tile_m, hidden), row(tile_m, hidden)],
        scratch_shapes=[pltpu.VMEM((tile_m, hidden), _F32)],
        compiler_params=cparams,
    )(a_ind, xwl, invdeg, self1, wl2_bf, wr2_bf, params["bl2"])

    out = pl.pallas_call(
        _layer2_kernel,
        out_shape=jax.ShapeDtypeStruct((n, 1), _F32),
        grid=grid,
        in_specs=[
            a_spec,
            panel(tile_k, hidden),
            row(tile_m, 1),
            row(tile_m, hidden),
            const((1, hidden)),
            const((1, 1)),
        ],
        out_specs=row(tile_m, 1),
        scratch_shapes=[pltpu.VMEM((tile_m, hidden), _F32)],
        compiler_params=cparams,
    )(a_ind, h1w, invdeg, h1r, params["wo"].T, params["bo"])

    return out[:, 0]


def kernel(embed, wt1, bt1, wt2, bt2, wl1, bl1, wr1, wl2, bl2, wr2, wo, bo,
           x, edge_index, t):
    params = {
        "embed": embed, "wt1": wt1, "bt1": bt1, "wt2": wt2, "bt2": bt2,
        "wl1": wl1, "bl1": bl1, "wr1": wr1, "wl2": wl2, "bl2": bl2,
        "wr2": wr2, "wo": wo, "bo": bo,
    }
    return _forward(params, x, edge_index, t, num_nodes=x.shape[0])
